# Initial kernel scaffold; baseline (speedup 1.0000x reference)
import jax
import jax.numpy as jnp
from jax.experimental import pallas as pl


def kernel(x, edge_index, W1, b1, W2, b2):
    raise NotImplementedError("write your pallas kernel here")


# speedup vs baseline: 10.8415x; 10.8415x over previous
import functools

import jax
import jax.numpy as jnp
from jax import lax
from jax.experimental import pallas as pl
from jax.experimental.pallas import tpu as pltpu
from jax.experimental.pallas import tpu_sc as plsc

N = 10000
E = 160000
F_IN = 256
HID = 256
CLS = 64

NPAD = 10240
K = 128
EPAD = 32 * 40 * K
ROWS = EPAD // K
RB = 10
BLK = NPAD // RB


def _mesh():
  return plsc.VectorSubcoreMesh(
      core_axis_name="c", subcore_axis_name="s", num_cores=2, num_subcores=16)


def _deg_body(dst_hbm, out_hbm, idx_v, ones_v, zb_v, wout_v, hist_sh):
  c = lax.axis_index("c")
  s = lax.axis_index("s")
  wid = c * 16 + s
  ones16 = jnp.ones((16,), jnp.float32)
  zeros16 = jnp.zeros((16,), jnp.float32)
  for i in range(8):
    ones_v[pl.ds(i * 16, 16)] = ones16
    zb_v[pl.ds(i * 16, 16)] = zeros16
  for i in range(5):
    pltpu.sync_copy(zb_v, hist_sh.at[pl.ds(s * 640 + i * 128, 128)])
  plsc.subcore_barrier()

  @pl.loop(0, 40)
  def _(j):
    row = wid * 40 + j
    pltpu.sync_copy(dst_hbm.at[row], idx_v)
    pltpu.sync_copy(ones_v, hist_sh.at[idx_v], add=True)

  plsc.subcore_barrier()
  pltpu.sync_copy(hist_sh.at[pl.ds(s * 640, 640)], wout_v)
  pltpu.sync_copy(wout_v, out_hbm.at[c, pl.ds(s * 640, 640)])


def _deg_call(dst2d):
  return pl.kernel(
      _deg_body,
      out_type=jax.ShapeDtypeStruct((2, NPAD), jnp.float32),
      mesh=_mesh(),
      scratch_types=[
          pltpu.VMEM((K,), jnp.int32),
          pltpu.VMEM((K,), jnp.float32),
          pltpu.VMEM((K,), jnp.float32),
          pltpu.VMEM((640,), jnp.float32),
          pltpu.VMEM_SHARED((NPAD,), jnp.float32),
      ],
  )(dst2d)


def _a1_body(yf_hbm, src_hbm, dst_hbm, out_hbm, sidx_v, didx_v, rows_v, acc_sh):
  c = lax.axis_index("c")
  s = lax.axis_index("s")
  base = c * NPAD

  @pl.loop(0, 5)
  def _(i):
    r0 = s * 640 + i * 128
    pltpu.sync_copy(yf_hbm.at[pl.ds(base + r0, 128), :], rows_v)
    pltpu.sync_copy(rows_v, acc_sh.at[pl.ds(r0, 128), :])

  plsc.subcore_barrier()

  @pl.loop(0, 80)
  def _(j):
    row = s * 80 + j
    pltpu.sync_copy(src_hbm.at[row], sidx_v)
    pltpu.sync_copy(dst_hbm.at[row], didx_v)
    for i in range(8):
      sidx_v[pl.ds(i * 16, 16)] = sidx_v[pl.ds(i * 16, 16)] + base
    pltpu.sync_copy(yf_hbm.at[sidx_v], rows_v)
    pltpu.sync_copy(rows_v, acc_sh.at[didx_v], add=True)

  plsc.subcore_barrier()

  @pl.loop(0, 5)
  def _(i):
    r0 = s * 640 + i * 128
    pltpu.sync_copy(acc_sh.at[pl.ds(r0, 128), :], rows_v)
    pltpu.sync_copy(rows_v, out_hbm.at[c, pl.ds(r0, 128), :])


def _a1_call(yf, src2d, dst2d):
  return pl.kernel(
      _a1_body,
      out_type=jax.ShapeDtypeStruct((2, NPAD, 128), jnp.float32),
      mesh=_mesh(),
      scratch_types=[
          pltpu.VMEM((K,), jnp.int32),
          pltpu.VMEM((K,), jnp.int32),
          pltpu.VMEM((K, 128), jnp.float32),
          pltpu.VMEM_SHARED((NPAD, 128), jnp.float32),
      ],
  )(yf, src2d, dst2d)


def _a2_body(y2_hbm, src_hbm, dst_hbm, zero_hbm, out_hbm,
             sidx_v, didx_v, rows_v, acc_sh):
  c = lax.axis_index("c")
  s = lax.axis_index("s")
  wid = c * 16 + s

  @pl.loop(0, 5)
  def _(i):
    r0 = s * 640 + i * 128

    @pl.when(c == 0)
    def _():
      pltpu.sync_copy(y2_hbm.at[pl.ds(r0, 128), :], rows_v)

    @pl.when(c == 1)
    def _():
      pltpu.sync_copy(zero_hbm.at[pl.ds(r0, 128), :], rows_v)

    pltpu.sync_copy(rows_v, acc_sh.at[pl.ds(r0, 128), :])

  plsc.subcore_barrier()

  @pl.loop(0, 40)
  def _(j):
    row = wid * 40 + j
    pltpu.sync_copy(src_hbm.at[row], sidx_v)
    pltpu.sync_copy(dst_hbm.at[row], didx_v)
    pltpu.sync_copy(y2_hbm.at[sidx_v], rows_v)
    pltpu.sync_copy(rows_v, acc_sh.at[didx_v], add=True)

  plsc.subcore_barrier()

  @pl.loop(0, 5)
  def _(i):
    r0 = s * 640 + i * 128
    pltpu.sync_copy(acc_sh.at[pl.ds(r0, 128), :], rows_v)
    pltpu.sync_copy(rows_v, out_hbm.at[c, pl.ds(r0, 128), :])


def _a2_call(y2, src2d, dst2d, zeros):
  return pl.kernel(
      _a2_body,
      out_type=jax.ShapeDtypeStruct((2, NPAD, 128), jnp.float32),
      mesh=_mesh(),
      scratch_types=[
          pltpu.VMEM((K,), jnp.int32),
          pltpu.VMEM((K,), jnp.int32),
          pltpu.VMEM((K, 128), jnp.float32),
          pltpu.VMEM_SHARED((NPAD, 128), jnp.float32),
      ],
  )(y2, src2d, dst2d, zeros)


def _m1_body(x_ref, w_ref, degT_ref, y_ref, dis_ref):
  xw = lax.dot_general(x_ref[...], w_ref[...], (((1,), (0,)), ((), ())),
                       precision=lax.Precision.HIGHEST,
                       preferred_element_type=jnp.float32)
  deg = degT_ref[:, 0:1] + degT_ref[:, 1:2] + 1.0
  dis = lax.rsqrt(deg)
  y = xw * dis
  y_ref[0] = y[:, :128]
  y_ref[1] = y[:, 128:]
  dis_ref[...] = dis


def _m1_call(xp, W1, degT):
  return pl.pallas_call(
      _m1_body,
      grid=(RB,),
      in_specs=[
          pl.BlockSpec((BLK, F_IN), lambda i: (i, 0)),
          pl.BlockSpec((F_IN, HID), lambda i: (0, 0)),
          pl.BlockSpec((BLK, 2), lambda i: (i, 0)),
      ],
      out_specs=[
          pl.BlockSpec((2, BLK, 128), lambda i: (0, i, 0)),
          pl.BlockSpec((BLK, 1), lambda i: (i, 0)),
      ],
      out_shape=[
          jax.ShapeDtypeStruct((2, NPAD, 128), jnp.float32),
          jax.ShapeDtypeStruct((NPAD, 1), jnp.float32),
      ],
  )(xp, W1, degT)


def _m2_body(z_ref, dis_ref, b1_ref, w2_ref, y2_ref):
  cat = jnp.concatenate([z_ref[0], z_ref[1]], axis=1)
  dis = dis_ref[...]
  h = jnp.maximum(cat * dis + b1_ref[...], 0.0)
  y2 = lax.dot_general(h, w2_ref[...], (((1,), (0,)), ((), ())),
                       precision=lax.Precision.HIGHEST,
                       preferred_element_type=jnp.float32)
  y2_ref[:, :CLS] = y2 * dis
  y2_ref[:, CLS:] = jnp.zeros((BLK, 128 - CLS), jnp.float32)


def _m2_call(z1, dis, b1r, W2):
  return pl.pallas_call(
      _m2_body,
      grid=(RB,),
      in_specs=[
          pl.BlockSpec((2, BLK, 128), lambda i: (0, i, 0)),
          pl.BlockSpec((BLK, 1), lambda i: (i, 0)),
          pl.BlockSpec((1, HID), lambda i: (0, 0)),
          pl.BlockSpec((HID, CLS), lambda i: (0, 0)),
      ],
      out_specs=pl.BlockSpec((BLK, 128), lambda i: (i, 0)),
      out_shape=jax.ShapeDtypeStruct((NPAD, 128), jnp.float32),
  )(z1, dis, b1r, W2)


def _m3_body(p_ref, dis_ref, b2_ref, o_ref):
  z = p_ref[0, :, :CLS] + p_ref[1, :, :CLS]
  o = z * dis_ref[...] + b2_ref[...]
  m = jnp.max(o, axis=1, keepdims=True)
  e = jnp.exp(o - m)
  lse = jnp.log(jnp.sum(e, axis=1, keepdims=True)) + m
  o_ref[...] = o - lse


def _m3_call(parts, dis, b2r):
  return pl.pallas_call(
      _m3_body,
      grid=(RB,),
      in_specs=[
          pl.BlockSpec((2, BLK, 128), lambda i: (0, i, 0)),
          pl.BlockSpec((BLK, 1), lambda i: (i, 0)),
          pl.BlockSpec((1, CLS), lambda i: (0, 0)),
      ],
      out_specs=pl.BlockSpec((BLK, CLS), lambda i: (i, 0)),
      out_shape=jax.ShapeDtypeStruct((NPAD, CLS), jnp.float32),
  )(parts, dis, b2r)


def kernel(x, edge_index, W1, b1, W2, b2):
  src = edge_index[0]
  dst = edge_index[1]
  padi = (jnp.arange(EPAD - E, dtype=jnp.int32) % (NPAD - N)) + N
  src2d = jnp.concatenate([src, padi]).reshape(ROWS, K)
  dst2d = jnp.concatenate([dst, padi]).reshape(ROWS, K)
  xp = jnp.concatenate([x, jnp.zeros((NPAD - N, F_IN), x.dtype)], axis=0)

  degs = _deg_call(dst2d)
  degT = degs.T
  y1, dis = _m1_call(xp, W1, degT)
  z1 = _a1_call(y1.reshape(2 * NPAD, 128), src2d, dst2d)
  y2 = _m2_call(z1, dis, b1.reshape(1, HID), W2)
  zeros = jnp.zeros((NPAD, 128), jnp.float32)
  parts = _a2_call(y2, src2d, dst2d, zeros)
  out = _m3_call(parts, dis, b2.reshape(1, CLS))
  return out[:N]


# speedup vs baseline: 17.7951x; 1.6414x over previous
import functools

import jax
import jax.numpy as jnp
from jax import lax
from jax.experimental import pallas as pl
from jax.experimental.pallas import tpu as pltpu
from jax.experimental.pallas import tpu_sc as plsc

N = 10000
E = 160000
F_IN = 256
HID = 256
CLS = 64

NPAD = 10240
K = 128
EPAD = 32 * 40 * K
ROWS = EPAD // K
RB = 10
BLK = NPAD // RB


def _mesh():
  return plsc.VectorSubcoreMesh(
      core_axis_name="c", subcore_axis_name="s", num_cores=2, num_subcores=16)


def _deg_body(dst_hbm, out_hbm, idx_v, ones_v, zb_v, wout_v, hist_sh):
  c = lax.axis_index("c")
  s = lax.axis_index("s")
  wid = c * 16 + s
  ones16 = jnp.ones((16,), jnp.float32)
  zeros16 = jnp.zeros((16,), jnp.float32)
  for i in range(8):
    ones_v[pl.ds(i * 16, 16)] = ones16
    zb_v[pl.ds(i * 16, 16)] = zeros16
  for i in range(5):
    pltpu.sync_copy(zb_v, hist_sh.at[pl.ds(s * 640 + i * 128, 128)])
  plsc.subcore_barrier()

  @pl.loop(0, 40)
  def _(j):
    row = wid * 40 + j
    pltpu.sync_copy(dst_hbm.at[row], idx_v)
    pltpu.sync_copy(ones_v, hist_sh.at[idx_v], add=True)

  plsc.subcore_barrier()
  pltpu.sync_copy(hist_sh.at[pl.ds(s * 640, 640)], wout_v)
  pltpu.sync_copy(wout_v, out_hbm.at[c, pl.ds(s * 640, 640)])


def _deg_call(dst2d):
  return pl.kernel(
      _deg_body,
      out_type=jax.ShapeDtypeStruct((2, NPAD), jnp.float32),
      mesh=_mesh(),
      scratch_types=[
          pltpu.VMEM((K,), jnp.int32),
          pltpu.VMEM((K,), jnp.float32),
          pltpu.VMEM((K,), jnp.float32),
          pltpu.VMEM((640,), jnp.float32),
          pltpu.VMEM_SHARED((NPAD,), jnp.float32),
      ],
  )(dst2d)


def _a1_body(yf_hbm, eidx_hbm, out_hbm, idx_v, rows_v, acc_sh, sem0, sem1):
  c = lax.axis_index("c")
  s = lax.axis_index("s")
  base = c * NPAD
  sems = (sem0, sem1)
  nb = 80

  pltpu.sync_copy(yf_hbm.at[pl.ds(base + s * 640, 640), :],
                  acc_sh.at[pl.ds(s * 640, 640), :])
  plsc.subcore_barrier()

  def stage_and_fire(j, b):
    pltpu.sync_copy(eidx_hbm.at[s * nb + j], idx_v.at[b])
    for i in range(8):
      idx_v[b, 0, pl.ds(i * 16, 16)] = idx_v[b, 0, pl.ds(i * 16, 16)] + base
    pltpu.async_copy(yf_hbm.at[idx_v.at[b, 0]], rows_v.at[b], sems[b])

  def wait_and_scatter(b):
    pltpu.make_async_copy(yf_hbm.at[idx_v.at[b, 0]], rows_v.at[b],
                          sems[b]).wait()
    pltpu.sync_copy(rows_v.at[b], acc_sh.at[idx_v.at[b, 1]], add=True)

  stage_and_fire(0, 0)
  stage_and_fire(1, 1)

  @pl.loop(0, nb - 2, step=2)
  def _(j0):
    for b in range(2):
      wait_and_scatter(b)
      stage_and_fire(j0 + b + 2, b)

  for b in range(2):
    wait_and_scatter(b)

  plsc.subcore_barrier()
  pltpu.sync_copy(acc_sh.at[pl.ds(s * 640, 640), :],
                  out_hbm.at[c, pl.ds(s * 640, 640), :])


def _a1_call(yf, eidx):
  return pl.kernel(
      _a1_body,
      out_type=jax.ShapeDtypeStruct((2, NPAD, 128), jnp.float32),
      mesh=_mesh(),
      scratch_types=[
          pltpu.VMEM((2, 2, K), jnp.int32),
          pltpu.VMEM((2, K, 128), jnp.float32),
          pltpu.VMEM_SHARED((NPAD, 128), jnp.float32),
          pltpu.SemaphoreType.DMA,
          pltpu.SemaphoreType.DMA,
      ],
  )(yf, eidx)


def _a2_body(y2_hbm, eidx_hbm, zero_hbm, out_hbm, idx_v, rows_v, acc_sh,
             sem0, sem1):
  c = lax.axis_index("c")
  s = lax.axis_index("s")
  wid = c * 16 + s
  sems = (sem0, sem1)
  nb = 40

  @pl.when(c == 0)
  def _():
    pltpu.sync_copy(y2_hbm.at[pl.ds(s * 640, 640), :],
                    acc_sh.at[pl.ds(s * 640, 640), :])

  @pl.when(c == 1)
  def _():
    pltpu.sync_copy(zero_hbm.at[pl.ds(s * 640, 640), :],
                    acc_sh.at[pl.ds(s * 640, 640), :])

  plsc.subcore_barrier()

  def stage_and_fire(j, b):
    pltpu.sync_copy(eidx_hbm.at[wid * nb + j], idx_v.at[b])
    pltpu.async_copy(y2_hbm.at[idx_v.at[b, 0]], rows_v.at[b], sems[b])

  def wait_and_scatter(b):
    pltpu.make_async_copy(y2_hbm.at[idx_v.at[b, 0]], rows_v.at[b],
                          sems[b]).wait()
    pltpu.sync_copy(rows_v.at[b], acc_sh.at[idx_v.at[b, 1]], add=True)

  stage_and_fire(0, 0)
  stage_and_fire(1, 1)

  @pl.loop(0, nb - 2, step=2)
  def _(j0):
    for b in range(2):
      wait_and_scatter(b)
      stage_and_fire(j0 + b + 2, b)

  for b in range(2):
    wait_and_scatter(b)

  plsc.subcore_barrier()
  pltpu.sync_copy(acc_sh.at[pl.ds(s * 640, 640), :],
                  out_hbm.at[c, pl.ds(s * 640, 640), :])


def _a2_call(y2, eidx, zeros):
  return pl.kernel(
      _a2_body,
      out_type=jax.ShapeDtypeStruct((2, NPAD, 128), jnp.float32),
      mesh=_mesh(),
      scratch_types=[
          pltpu.VMEM((2, 2, K), jnp.int32),
          pltpu.VMEM((2, K, 128), jnp.float32),
          pltpu.VMEM_SHARED((NPAD, 128), jnp.float32),
          pltpu.SemaphoreType.DMA,
          pltpu.SemaphoreType.DMA,
      ],
  )(y2, eidx, zeros)


def _m1_body(x_ref, w_ref, degT_ref, y_ref, dis_ref):
  xw = lax.dot_general(x_ref[...], w_ref[...], (((1,), (0,)), ((), ())),
                       precision=lax.Precision.HIGHEST,
                       preferred_element_type=jnp.float32)
  deg = degT_ref[:, 0:1] + degT_ref[:, 1:2] + 1.0
  dis = lax.rsqrt(deg)
  y = xw * dis
  y_ref[0] = y[:, :128]
  y_ref[1] = y[:, 128:]
  dis_ref[...] = dis


def _m1_call(xp, W1, degT):
  return pl.pallas_call(
      _m1_body,
      grid=(RB,),
      in_specs=[
          pl.BlockSpec((BLK, F_IN), lambda i: (i, 0)),
          pl.BlockSpec((F_IN, HID), lambda i: (0, 0)),
          pl.BlockSpec((BLK, 2), lambda i: (i, 0)),
      ],
      out_specs=[
          pl.BlockSpec((2, BLK, 128), lambda i: (0, i, 0)),
          pl.BlockSpec((BLK, 1), lambda i: (i, 0)),
      ],
      out_shape=[
          jax.ShapeDtypeStruct((2, NPAD, 128), jnp.float32),
          jax.ShapeDtypeStruct((NPAD, 1), jnp.float32),
      ],
  )(xp, W1, degT)


def _m2_body(z_ref, dis_ref, b1_ref, w2_ref, y2_ref):
  cat = jnp.concatenate([z_ref[0], z_ref[1]], axis=1)
  dis = dis_ref[...]
  h = jnp.maximum(cat * dis + b1_ref[...], 0.0)
  y2 = lax.dot_general(h, w2_ref[...], (((1,), (0,)), ((), ())),
                       precision=lax.Precision.HIGHEST,
                       preferred_element_type=jnp.float32)
  y2_ref[:, :CLS] = y2 * dis
  y2_ref[:, CLS:] = jnp.zeros((BLK, 128 - CLS), jnp.float32)


def _m2_call(z1, dis, b1r, W2):
  return pl.pallas_call(
      _m2_body,
      grid=(RB,),
      in_specs=[
          pl.BlockSpec((2, BLK, 128), lambda i: (0, i, 0)),
          pl.BlockSpec((BLK, 1), lambda i: (i, 0)),
          pl.BlockSpec((1, HID), lambda i: (0, 0)),
          pl.BlockSpec((HID, CLS), lambda i: (0, 0)),
      ],
      out_specs=pl.BlockSpec((BLK, 128), lambda i: (i, 0)),
      out_shape=jax.ShapeDtypeStruct((NPAD, 128), jnp.float32),
  )(z1, dis, b1r, W2)


def _m3_body(p_ref, dis_ref, b2_ref, o_ref):
  z = p_ref[0, :, :CLS] + p_ref[1, :, :CLS]
  o = z * dis_ref[...] + b2_ref[...]
  m = jnp.max(o, axis=1, keepdims=True)
  e = jnp.exp(o - m)
  lse = jnp.log(jnp.sum(e, axis=1, keepdims=True)) + m
  o_ref[...] = o - lse


def _m3_call(parts, dis, b2r):
  return pl.pallas_call(
      _m3_body,
      grid=(RB,),
      in_specs=[
          pl.BlockSpec((2, BLK, 128), lambda i: (0, i, 0)),
          pl.BlockSpec((BLK, 1), lambda i: (i, 0)),
          pl.BlockSpec((1, CLS), lambda i: (0, 0)),
      ],
      out_specs=pl.BlockSpec((BLK, CLS), lambda i: (i, 0)),
      out_shape=jax.ShapeDtypeStruct((NPAD, CLS), jnp.float32),
  )(parts, dis, b2r)


def kernel(x, edge_index, W1, b1, W2, b2):
  src = edge_index[0]
  dst = edge_index[1]
  padi = (jnp.arange(EPAD - E, dtype=jnp.int32) % (NPAD - N)) + N
  src2d = jnp.concatenate([src, padi]).reshape(ROWS, K)
  dst2d = jnp.concatenate([dst, padi]).reshape(ROWS, K)
  eidx = jnp.stack([src2d, dst2d], axis=1)
  xp = jnp.concatenate([x, jnp.zeros((NPAD - N, F_IN), x.dtype)], axis=0)

  degs = _deg_call(dst2d)
  degT = degs.T
  y1, dis = _m1_call(xp, W1, degT)
  z1 = _a1_call(y1.reshape(2 * NPAD, 128), eidx)
  y2 = _m2_call(z1, dis, b1.reshape(1, HID), W2)
  zeros = jnp.zeros((NPAD, 128), jnp.float32)
  parts = _a2_call(y2, eidx, zeros)
  out = _m3_call(parts, dis, b2.reshape(1, CLS))
  return out[:N]


# speedup vs baseline: 18.5261x; 1.0411x over previous
import functools

import jax
import jax.numpy as jnp
from jax import lax
from jax.experimental import pallas as pl
from jax.experimental.pallas import tpu as pltpu
from jax.experimental.pallas import tpu_sc as plsc

N = 10000
E = 160000
F_IN = 256
HID = 256
CLS = 64

NPAD = 10112
TSLC = NPAD // 16
K = 128
EPAD = 32 * 40 * K
ROWS = EPAD // K
RB = 8
BLK = NPAD // RB


def _mesh():
  return plsc.VectorSubcoreMesh(
      core_axis_name="c", subcore_axis_name="s", num_cores=2, num_subcores=16)


def _deg_body(dst_hbm, out_hbm, idx_v, ones_v, zb_v, wout_v, hist_sh):
  c = lax.axis_index("c")
  s = lax.axis_index("s")
  wid = c * 16 + s
  ones16 = jnp.ones((16,), jnp.float32)
  zeros16 = jnp.zeros((16,), jnp.float32)
  for i in range(8):
    ones_v[pl.ds(i * 16, 16)] = ones16
    zb_v[pl.ds(i * 16, 16)] = zeros16
  for i in range(5):
    pltpu.sync_copy(zb_v, hist_sh.at[pl.ds(s * 640 + i * 128, 128)])
  plsc.subcore_barrier()

  @pl.loop(0, 40)
  def _(j):
    row = wid * 40 + j
    pltpu.sync_copy(dst_hbm.at[row], idx_v)
    pltpu.sync_copy(ones_v, hist_sh.at[idx_v], add=True)

  plsc.subcore_barrier()
  pltpu.sync_copy(hist_sh.at[pl.ds(s * TSLC, TSLC)], wout_v)
  pltpu.sync_copy(wout_v, out_hbm.at[pl.ds(c * NPAD + s * TSLC, TSLC)])


def _deg_call(dst2d):
  return pl.kernel(
      _deg_body,
      out_type=jax.ShapeDtypeStruct((2 * NPAD,), jnp.float32),
      mesh=_mesh(),
      scratch_types=[
          pltpu.VMEM((K,), jnp.int32),
          pltpu.VMEM((K,), jnp.float32),
          pltpu.VMEM((K,), jnp.float32),
          pltpu.VMEM((TSLC,), jnp.float32),
          pltpu.VMEM_SHARED((10240,), jnp.float32),
      ],
  )(dst2d)


def _agg_body(tab_hbm, eidx_hbm, out_hbm, idx_v, rows_v, acc_sh, gsems, ssems,
              *, nb, row0_fn, base_fn, init_fn):
  c = lax.axis_index("c")
  s = lax.axis_index("s")
  base = base_fn(c)
  row0 = row0_fn(c, s)

  init_fn(c, s, acc_sh)
  plsc.subcore_barrier()

  def fire_gather(j, b):
    pltpu.sync_copy(eidx_hbm.at[row0 + j], idx_v.at[b])
    if base is not None:
      for i in range(8):
        idx_v[b, 0, pl.ds(i * 16, 16)] = idx_v[b, 0, pl.ds(i * 16, 16)] + base
    pltpu.async_copy(tab_hbm.at[idx_v.at[b, 0]], rows_v.at[b], gsems[b])

  def wait_gather(b):
    pltpu.make_async_copy(tab_hbm.at[idx_v.at[b, 0]], rows_v.at[b],
                          gsems[b]).wait()

  def fire_scatter(b):
    pltpu.async_copy(rows_v.at[b], acc_sh.at[idx_v.at[b, 1]], ssems[b],
                     add=True)

  def wait_scatter(b):
    pltpu.make_async_copy(rows_v.at[b], acc_sh.at[idx_v.at[b, 1]],
                          ssems[b]).wait()

  fire_gather(0, 0)
  fire_gather(1, 1)
  fire_gather(2, 2)
  wait_gather(0)
  fire_scatter(0)

  main_iters = ((nb - 3) // 3) * 3

  @pl.loop(1, 1 + main_iters, step=3)
  def _(j0):
    for t in range(3):
      b = (1 + t) % 3
      bg = (t % 3)
      wait_scatter(bg)
      fire_gather(j0 + t + 2, bg)
      wait_gather(b)
      fire_scatter(b)

  for j in range(1 + main_iters, nb - 2):
    wait_scatter((j + 2) % 3)
    fire_gather(j + 2, (j + 2) % 3)
    wait_gather(j % 3)
    fire_scatter(j % 3)

  for j in (nb - 2, nb - 1):
    wait_scatter((j + 2) % 3)
    wait_gather(j % 3)
    fire_scatter(j % 3)
  wait_scatter((nb - 1) % 3)

  plsc.subcore_barrier()
  pltpu.sync_copy(acc_sh.at[pl.ds(s * TSLC, TSLC), :],
                  out_hbm.at[c, pl.ds(s * TSLC, TSLC), :])


def _agg_scratch():
  return [
      pltpu.VMEM((3, 2, K), jnp.int32),
      pltpu.VMEM((3, K, 128), jnp.float32),
      pltpu.VMEM_SHARED((NPAD, 128), jnp.float32),
      pltpu.SemaphoreType.DMA,
      pltpu.SemaphoreType.DMA,
      pltpu.SemaphoreType.DMA,
      pltpu.SemaphoreType.DMA,
      pltpu.SemaphoreType.DMA,
      pltpu.SemaphoreType.DMA,
  ]


def _a1_body(yf_hbm, eidx_hbm, out_hbm, idx_v, rows_v, acc_sh,
             g0, g1, g2, s0, s1, s2):
  def init(c, s, acc_sh):
    pltpu.sync_copy(yf_hbm.at[pl.ds(c * NPAD + s * TSLC, TSLC), :],
                    acc_sh.at[pl.ds(s * TSLC, TSLC), :])

  _agg_body(yf_hbm, eidx_hbm, out_hbm, idx_v, rows_v, acc_sh,
            (g0, g1, g2), (s0, s1, s2),
            nb=80, row0_fn=lambda c, s: s * 80,
            base_fn=lambda c: c * NPAD, init_fn=init)


def _a1_call(yf, eidx):
  return pl.kernel(
      _a1_body,
      out_type=jax.ShapeDtypeStruct((2, NPAD, 128), jnp.float32),
      mesh=_mesh(),
      scratch_types=_agg_scratch(),
  )(yf, eidx)


def _a2_body(y2_hbm, eidx_hbm, zero_hbm, out_hbm, idx_v, rows_v, acc_sh,
             g0, g1, g2, s0, s1, s2):
  def init(c, s, acc_sh):
    @pl.when(c == 0)
    def _():
      pltpu.sync_copy(y2_hbm.at[pl.ds(s * TSLC, TSLC), :],
                      acc_sh.at[pl.ds(s * TSLC, TSLC), :])

    @pl.when(c == 1)
    def _():
      pltpu.sync_copy(zero_hbm.at[pl.ds(s * TSLC, TSLC), :],
                      acc_sh.at[pl.ds(s * TSLC, TSLC), :])

  _agg_body(y2_hbm, eidx_hbm, out_hbm, idx_v, rows_v, acc_sh,
            (g0, g1, g2), (s0, s1, s2),
            nb=40, row0_fn=lambda c, s: (c * 16 + s) * 40,
            base_fn=lambda c: None, init_fn=init)


def _a2_call(y2, eidx, zeros):
  return pl.kernel(
      _a2_body,
      out_type=jax.ShapeDtypeStruct((2, NPAD, 128), jnp.float32),
      mesh=_mesh(),
      scratch_types=_agg_scratch(),
  )(y2, eidx, zeros)


def _m1_body(x_ref, w_ref, degT_ref, y_ref, dis_ref):
  xw = lax.dot_general(x_ref[...], w_ref[...], (((1,), (0,)), ((), ())),
                       precision=lax.Precision.HIGHEST,
                       preferred_element_type=jnp.float32)
  deg = degT_ref[:, 0:1] + degT_ref[:, 1:2] + 1.0
  dis = lax.rsqrt(deg)
  y = xw * dis
  y_ref[0] = y[:, :128]
  y_ref[1] = y[:, 128:]
  dis_ref[...] = dis


def _m1_call(xp, W1, degT):
  return pl.pallas_call(
      _m1_body,
      grid=(RB,),
      in_specs=[
          pl.BlockSpec((BLK, F_IN), lambda i: (i, 0)),
          pl.BlockSpec((F_IN, HID), lambda i: (0, 0)),
          pl.BlockSpec((BLK, 2), lambda i: (i, 0)),
      ],
      out_specs=[
          pl.BlockSpec((2, BLK, 128), lambda i: (0, i, 0)),
          pl.BlockSpec((BLK, 1), lambda i: (i, 0)),
      ],
      out_shape=[
          jax.ShapeDtypeStruct((2, NPAD, 128), jnp.float32),
          jax.ShapeDtypeStruct((NPAD, 1), jnp.float32),
      ],
  )(xp, W1, degT)


def _m2_body(z_ref, dis_ref, b1_ref, w2_ref, y2_ref):
  cat = jnp.concatenate([z_ref[0], z_ref[1]], axis=1)
  dis = dis_ref[...]
  h = jnp.maximum(cat * dis + b1_ref[...], 0.0)
  y2 = lax.dot_general(h, w2_ref[...], (((1,), (0,)), ((), ())),
                       precision=lax.Precision.HIGHEST,
                       preferred_element_type=jnp.float32)
  y2_ref[:, :CLS] = y2 * dis
  y2_ref[:, CLS:] = jnp.zeros((BLK, 128 - CLS), jnp.float32)


def _m2_call(z1, dis, b1r, W2):
  return pl.pallas_call(
      _m2_body,
      grid=(RB,),
      in_specs=[
          pl.BlockSpec((2, BLK, 128), lambda i: (0, i, 0)),
          pl.BlockSpec((BLK, 1), lambda i: (i, 0)),
          pl.BlockSpec((1, HID), lambda i: (0, 0)),
          pl.BlockSpec((HID, CLS), lambda i: (0, 0)),
      ],
      out_specs=pl.BlockSpec((BLK, 128), lambda i: (i, 0)),
      out_shape=jax.ShapeDtypeStruct((NPAD, 128), jnp.float32),
  )(z1, dis, b1r, W2)


def _m3_body(p_ref, dis_ref, b2_ref, o_ref):
  z = p_ref[0, :, :CLS] + p_ref[1, :, :CLS]
  o = z * dis_ref[...] + b2_ref[...]
  m = jnp.max(o, axis=1, keepdims=True)
  e = jnp.exp(o - m)
  lse = jnp.log(jnp.sum(e, axis=1, keepdims=True)) + m
  o_ref[...] = o - lse


def _m3_call(parts, dis, b2r):
  return pl.pallas_call(
      _m3_body,
      grid=(RB,),
      in_specs=[
          pl.BlockSpec((2, BLK, 128), lambda i: (0, i, 0)),
          pl.BlockSpec((BLK, 1), lambda i: (i, 0)),
          pl.BlockSpec((1, CLS), lambda i: (0, 0)),
      ],
      out_specs=pl.BlockSpec((BLK, CLS), lambda i: (i, 0)),
      out_shape=jax.ShapeDtypeStruct((NPAD, CLS), jnp.float32),
  )(parts, dis, b2r)


def kernel(x, edge_index, W1, b1, W2, b2):
  src = edge_index[0]
  dst = edge_index[1]
  padi = (jnp.arange(EPAD - E, dtype=jnp.int32) % (NPAD - N)) + N
  src2d = jnp.concatenate([src, padi]).reshape(ROWS, K)
  dst2d = jnp.concatenate([dst, padi]).reshape(ROWS, K)
  eidx = jnp.stack([src2d, dst2d], axis=1)
  xp = jnp.concatenate([x, jnp.zeros((NPAD - N, F_IN), x.dtype)], axis=0)

  degs = _deg_call(dst2d).reshape(2, NPAD)
  degT = degs.T
  y1, dis = _m1_call(xp, W1, degT)
  z1 = _a1_call(y1.reshape(2 * NPAD, 128), eidx)
  y2 = _m2_call(z1, dis, b1.reshape(1, HID), W2)
  zeros = jnp.zeros((NPAD, 128), jnp.float32)
  parts = _a2_call(y2, eidx, zeros)
  out = _m3_call(parts, dis, b2.reshape(1, CLS))
  return out[:N]


# speedup vs baseline: 18.9833x; 1.0247x over previous
import functools

import jax
import jax.numpy as jnp
from jax import lax
from jax.experimental import pallas as pl
from jax.experimental.pallas import tpu as pltpu
from jax.experimental.pallas import tpu_sc as plsc

N = 10000
E = 160000
F_IN = 256
HID = 256
CLS = 64

NPAD = 10112
TSLC = NPAD // 16
K = 128
EPAD = 32 * 40 * K
ROWS = EPAD // K
RB = 8
BLK = NPAD // RB


def _mesh():
  return plsc.VectorSubcoreMesh(
      core_axis_name="c", subcore_axis_name="s", num_cores=2, num_subcores=16)


def _deg_body(dst_hbm, out_hbm, idx_v, ones_v, zb_v, wout_v, hist_sh):
  c = lax.axis_index("c")
  s = lax.axis_index("s")
  wid = c * 16 + s
  ones16 = jnp.ones((16,), jnp.float32)
  zeros16 = jnp.zeros((16,), jnp.float32)
  for i in range(8):
    ones_v[pl.ds(i * 16, 16)] = ones16
    zb_v[pl.ds(i * 16, 16)] = zeros16
  for i in range(5):
    pltpu.sync_copy(zb_v, hist_sh.at[pl.ds(s * 640 + i * 128, 128)])
  plsc.subcore_barrier()

  @pl.loop(0, 40)
  def _(j):
    row = wid * 40 + j
    pltpu.sync_copy(dst_hbm.at[row], idx_v)
    pltpu.sync_copy(ones_v, hist_sh.at[idx_v], add=True)

  plsc.subcore_barrier()
  pltpu.sync_copy(hist_sh.at[pl.ds(s * TSLC, TSLC)], wout_v)
  pltpu.sync_copy(wout_v, out_hbm.at[pl.ds(c * NPAD + s * TSLC, TSLC)])


def _deg_call(dst2d):
  return pl.kernel(
      _deg_body,
      out_type=jax.ShapeDtypeStruct((2 * NPAD,), jnp.float32),
      mesh=_mesh(),
      scratch_types=[
          pltpu.VMEM((K,), jnp.int32),
          pltpu.VMEM((K,), jnp.float32),
          pltpu.VMEM((K,), jnp.float32),
          pltpu.VMEM((TSLC,), jnp.float32),
          pltpu.VMEM_SHARED((10240,), jnp.float32),
      ],
  )(dst2d)


def _agg_body(tab_hbm, eidx_hbm, out_hbm, idx_v, rows_v, acc_sh, gsems, ssems,
              *, nb, row0_fn, base_fn, init_fn):
  c = lax.axis_index("c")
  s = lax.axis_index("s")
  base = base_fn(c)
  row0 = row0_fn(c, s)

  init_fn(c, s, acc_sh)
  plsc.subcore_barrier()

  def fire_gather(j, b):
    pltpu.sync_copy(eidx_hbm.at[row0 + j], idx_v.at[b])
    if base is not None:
      for i in range(8):
        idx_v[b, 0, pl.ds(i * 16, 16)] = idx_v[b, 0, pl.ds(i * 16, 16)] + base
    pltpu.async_copy(tab_hbm.at[idx_v.at[b, 0]], rows_v.at[b], gsems[b])

  def wait_gather(b):
    pltpu.make_async_copy(tab_hbm.at[idx_v.at[b, 0]], rows_v.at[b],
                          gsems[b]).wait()

  def fire_scatter(b):
    pltpu.async_copy(rows_v.at[b], acc_sh.at[idx_v.at[b, 1]], ssems[b],
                     add=True)

  def wait_scatter(b):
    pltpu.make_async_copy(rows_v.at[b], acc_sh.at[idx_v.at[b, 1]],
                          ssems[b]).wait()

  fire_gather(0, 0)
  fire_gather(1, 1)
  fire_gather(2, 2)
  wait_gather(0)
  fire_scatter(0)

  main_iters = ((nb - 3) // 3) * 3

  @pl.loop(1, 1 + main_iters, step=3)
  def _(j0):
    for t in range(3):
      b = (1 + t) % 3
      bg = (t % 3)
      wait_scatter(bg)
      fire_gather(j0 + t + 2, bg)
      wait_gather(b)
      fire_scatter(b)

  for j in range(1 + main_iters, nb - 2):
    wait_scatter((j + 2) % 3)
    fire_gather(j + 2, (j + 2) % 3)
    wait_gather(j % 3)
    fire_scatter(j % 3)

  for j in (nb - 2, nb - 1):
    wait_scatter((j + 2) % 3)
    wait_gather(j % 3)
    fire_scatter(j % 3)
  wait_scatter((nb - 1) % 3)

  plsc.subcore_barrier()
  pltpu.sync_copy(acc_sh.at[pl.ds(s * TSLC, TSLC), :],
                  out_hbm.at[c, pl.ds(s * TSLC, TSLC), :])


def _agg_scratch():
  return [
      pltpu.VMEM((3, 2, K), jnp.int32),
      pltpu.VMEM((3, K, 128), jnp.float32),
      pltpu.VMEM_SHARED((NPAD, 128), jnp.float32),
      pltpu.SemaphoreType.DMA,
      pltpu.SemaphoreType.DMA,
      pltpu.SemaphoreType.DMA,
      pltpu.SemaphoreType.DMA,
      pltpu.SemaphoreType.DMA,
      pltpu.SemaphoreType.DMA,
  ]


def _a1_body(yf_hbm, eidx_hbm, out_hbm, idx_v, rows_v, acc_sh,
             g0, g1, g2, s0, s1, s2):
  def init(c, s, acc_sh):
    pltpu.sync_copy(yf_hbm.at[pl.ds(c * NPAD + s * TSLC, TSLC), :],
                    acc_sh.at[pl.ds(s * TSLC, TSLC), :])

  _agg_body(yf_hbm, eidx_hbm, out_hbm, idx_v, rows_v, acc_sh,
            (g0, g1, g2), (s0, s1, s2),
            nb=80, row0_fn=lambda c, s: s * 80,
            base_fn=lambda c: c * NPAD, init_fn=init)


def _a1_call(yf, eidx):
  return pl.kernel(
      _a1_body,
      out_type=jax.ShapeDtypeStruct((2, NPAD, 128), jnp.float32),
      mesh=_mesh(),
      scratch_types=_agg_scratch(),
  )(yf, eidx)


def _a2_body(y2_hbm, eidx_hbm, zero_hbm, out_hbm, idx_v, rows_v, acc_sh,
             g0, g1, g2, s0, s1, s2):
  def init(c, s, acc_sh):
    @pl.when(c == 0)
    def _():
      pltpu.sync_copy(y2_hbm.at[pl.ds(s * TSLC, TSLC), :],
                      acc_sh.at[pl.ds(s * TSLC, TSLC), :])

    @pl.when(c == 1)
    def _():
      pltpu.sync_copy(zero_hbm.at[pl.ds(s * TSLC, TSLC), :],
                      acc_sh.at[pl.ds(s * TSLC, TSLC), :])

  _agg_body(y2_hbm, eidx_hbm, out_hbm, idx_v, rows_v, acc_sh,
            (g0, g1, g2), (s0, s1, s2),
            nb=40, row0_fn=lambda c, s: (c * 16 + s) * 40,
            base_fn=lambda c: None, init_fn=init)


def _a2_call(y2, eidx, zeros):
  return pl.kernel(
      _a2_body,
      out_type=jax.ShapeDtypeStruct((2, NPAD, CLS), jnp.float32),
      mesh=_mesh(),
      scratch_types=[
          pltpu.VMEM((3, 2, K), jnp.int32),
          pltpu.VMEM((3, K, CLS), jnp.float32),
          pltpu.VMEM_SHARED((NPAD, CLS), jnp.float32),
          pltpu.SemaphoreType.DMA,
          pltpu.SemaphoreType.DMA,
          pltpu.SemaphoreType.DMA,
          pltpu.SemaphoreType.DMA,
          pltpu.SemaphoreType.DMA,
          pltpu.SemaphoreType.DMA,
      ],
      compiler_params=pltpu.CompilerParams(use_tc_tiling_on_sc=False),
  )(y2, eidx, zeros)


def _m1_body(x_ref, w_ref, degT_ref, y_ref, dis_ref):
  xw = lax.dot_general(x_ref[...], w_ref[...], (((1,), (0,)), ((), ())),
                       precision=lax.Precision.HIGHEST,
                       preferred_element_type=jnp.float32)
  deg = degT_ref[:, 0:1] + degT_ref[:, 1:2] + 1.0
  dis = lax.rsqrt(deg)
  y = xw * dis
  y_ref[0] = y[:, :128]
  y_ref[1] = y[:, 128:]
  dis_ref[...] = dis


def _m1_call(xp, W1, degT):
  return pl.pallas_call(
      _m1_body,
      grid=(RB,),
      in_specs=[
          pl.BlockSpec((BLK, F_IN), lambda i: (i, 0)),
          pl.BlockSpec((F_IN, HID), lambda i: (0, 0)),
          pl.BlockSpec((BLK, 2), lambda i: (i, 0)),
      ],
      out_specs=[
          pl.BlockSpec((2, BLK, 128), lambda i: (0, i, 0)),
          pl.BlockSpec((BLK, 1), lambda i: (i, 0)),
      ],
      out_shape=[
          jax.ShapeDtypeStruct((2, NPAD, 128), jnp.float32),
          jax.ShapeDtypeStruct((NPAD, 1), jnp.float32),
      ],
  )(xp, W1, degT)


def _m2_body(z_ref, dis_ref, b1_ref, w2_ref, y2_ref):
  cat = jnp.concatenate([z_ref[0], z_ref[1]], axis=1)
  dis = dis_ref[...]
  h = jnp.maximum(cat * dis + b1_ref[...], 0.0)
  y2 = lax.dot_general(h, w2_ref[...], (((1,), (0,)), ((), ())),
                       precision=lax.Precision.HIGHEST,
                       preferred_element_type=jnp.float32)
  y2_ref[...] = y2 * dis


def _m2_call(z1, dis, b1r, W2):
  return pl.pallas_call(
      _m2_body,
      grid=(RB,),
      in_specs=[
          pl.BlockSpec((2, BLK, 128), lambda i: (0, i, 0)),
          pl.BlockSpec((BLK, 1), lambda i: (i, 0)),
          pl.BlockSpec((1, HID), lambda i: (0, 0)),
          pl.BlockSpec((HID, CLS), lambda i: (0, 0)),
      ],
      out_specs=pl.BlockSpec((BLK, CLS), lambda i: (i, 0)),
      out_shape=jax.ShapeDtypeStruct((NPAD, CLS), jnp.float32),
  )(z1, dis, b1r, W2)


def _m3_body(p_ref, dis_ref, b2_ref, o_ref):
  z = p_ref[0] + p_ref[1]
  o = z * dis_ref[...] + b2_ref[...]
  m = jnp.max(o, axis=1, keepdims=True)
  e = jnp.exp(o - m)
  lse = jnp.log(jnp.sum(e, axis=1, keepdims=True)) + m
  o_ref[...] = o - lse


def _m3_call(parts, dis, b2r):
  return pl.pallas_call(
      _m3_body,
      grid=(RB,),
      in_specs=[
          pl.BlockSpec((2, BLK, CLS), lambda i: (0, i, 0)),
          pl.BlockSpec((BLK, 1), lambda i: (i, 0)),
          pl.BlockSpec((1, CLS), lambda i: (0, 0)),
      ],
      out_specs=pl.BlockSpec((BLK, CLS), lambda i: (i, 0)),
      out_shape=jax.ShapeDtypeStruct((NPAD, CLS), jnp.float32),
  )(parts, dis, b2r)


def kernel(x, edge_index, W1, b1, W2, b2):
  src = edge_index[0]
  dst = edge_index[1]
  padi = (jnp.arange(EPAD - E, dtype=jnp.int32) % (NPAD - N)) + N
  src2d = jnp.concatenate([src, padi]).reshape(ROWS, K)
  dst2d = jnp.concatenate([dst, padi]).reshape(ROWS, K)
  eidx = jnp.stack([src2d, dst2d], axis=1)
  xp = jnp.concatenate([x, jnp.zeros((NPAD - N, F_IN), x.dtype)], axis=0)

  degs = _deg_call(dst2d).reshape(2, NPAD)
  degT = degs.T
  y1, dis = _m1_call(xp, W1, degT)
  z1 = _a1_call(y1.reshape(2 * NPAD, 128), eidx)
  y2 = _m2_call(z1, dis, b1.reshape(1, HID), W2)
  zeros = jnp.zeros((NPAD, CLS), jnp.float32)
  parts = _a2_call(y2, eidx, zeros)
  out = _m3_call(parts, dis, b2.reshape(1, CLS))
  return out[:N]


# speedup vs baseline: 19.2007x; 1.0115x over previous
import functools

import jax
import jax.numpy as jnp
from jax import lax
from jax.experimental import pallas as pl
from jax.experimental.pallas import tpu as pltpu
from jax.experimental.pallas import tpu_sc as plsc

N = 10000
E = 160000
F_IN = 256
HID = 256
CLS = 64

NPAD = 10112
TSLC = NPAD // 16
K = 128
EPAD = 32 * 40 * K
ROWS = EPAD // K
RB = 8
BLK = NPAD // RB


def _mesh():
  return plsc.VectorSubcoreMesh(
      core_axis_name="c", subcore_axis_name="s", num_cores=2, num_subcores=16)


def _deg_body(dst_hbm, out_hbm, idx_v, ones_v, zb_v, wout_v, hist_sh):
  c = lax.axis_index("c")
  s = lax.axis_index("s")
  wid = c * 16 + s
  ones16 = jnp.ones((16,), jnp.float32)
  zeros16 = jnp.zeros((16,), jnp.float32)
  for i in range(8):
    ones_v[pl.ds(i * 16, 16)] = ones16
    zb_v[pl.ds(i * 16, 16)] = zeros16
  for i in range(5):
    pltpu.sync_copy(zb_v, hist_sh.at[pl.ds(s * 640 + i * 128, 128)])
  plsc.subcore_barrier()

  @pl.loop(0, 40)
  def _(j):
    row = wid * 40 + j
    pltpu.sync_copy(dst_hbm.at[row], idx_v)
    pltpu.sync_copy(ones_v, hist_sh.at[idx_v], add=True)

  plsc.subcore_barrier()
  pltpu.sync_copy(hist_sh.at[pl.ds(s * TSLC, TSLC)], wout_v)
  pltpu.sync_copy(wout_v, out_hbm.at[pl.ds(c * NPAD + s * TSLC, TSLC)])


def _deg_call(dst2d):
  return pl.kernel(
      _deg_body,
      out_type=jax.ShapeDtypeStruct((2 * NPAD,), jnp.float32),
      mesh=_mesh(),
      scratch_types=[
          pltpu.VMEM((K,), jnp.int32),
          pltpu.VMEM((K,), jnp.float32),
          pltpu.VMEM((K,), jnp.float32),
          pltpu.VMEM((TSLC,), jnp.float32),
          pltpu.VMEM_SHARED((10240,), jnp.float32),
      ],
  )(dst2d)


def _agg_body(tab_hbm, eidx_hbm, out_hbm, idx_v, rows_v, acc_sh, gsems, ssems,
              *, nb, row0_fn, base_fn, init_fn):
  c = lax.axis_index("c")
  s = lax.axis_index("s")
  base = base_fn(c)
  row0 = row0_fn(c, s)

  init_fn(c, s, acc_sh)
  plsc.subcore_barrier()

  def fire_gather(j, b):
    pltpu.sync_copy(eidx_hbm.at[row0 + j], idx_v.at[b])
    if base is not None:
      for i in range(8):
        idx_v[b, 0, pl.ds(i * 16, 16)] = idx_v[b, 0, pl.ds(i * 16, 16)] + base
    pltpu.async_copy(tab_hbm.at[idx_v.at[b, 0]], rows_v.at[b], gsems[b])

  def wait_gather(b):
    pltpu.make_async_copy(tab_hbm.at[idx_v.at[b, 0]], rows_v.at[b],
                          gsems[b]).wait()

  def fire_scatter(b):
    pltpu.async_copy(rows_v.at[b], acc_sh.at[idx_v.at[b, 1]], ssems[b],
                     add=True)

  def wait_scatter(b):
    pltpu.make_async_copy(rows_v.at[b], acc_sh.at[idx_v.at[b, 1]],
                          ssems[b]).wait()

  fire_gather(0, 0)
  fire_gather(1, 1)
  fire_gather(2, 2)
  wait_gather(0)
  fire_scatter(0)

  main_iters = ((nb - 3) // 3) * 3

  @pl.loop(1, 1 + main_iters, step=3)
  def _(j0):
    for t in range(3):
      b = (1 + t) % 3
      bg = (t % 3)
      wait_scatter(bg)
      fire_gather(j0 + t + 2, bg)
      wait_gather(b)
      fire_scatter(b)

  for j in range(1 + main_iters, nb - 2):
    wait_scatter((j + 2) % 3)
    fire_gather(j + 2, (j + 2) % 3)
    wait_gather(j % 3)
    fire_scatter(j % 3)

  for j in (nb - 2, nb - 1):
    wait_scatter((j + 2) % 3)
    wait_gather(j % 3)
    fire_scatter(j % 3)
  wait_scatter((nb - 1) % 3)

  plsc.subcore_barrier()
  pltpu.sync_copy(acc_sh.at[pl.ds(s * TSLC, TSLC), :],
                  out_hbm.at[c, pl.ds(s * TSLC, TSLC), :])


def _agg_scratch():
  return [
      pltpu.VMEM((3, 2, K), jnp.int32),
      pltpu.VMEM((3, K, 128), jnp.float32),
      pltpu.VMEM_SHARED((NPAD, 128), jnp.float32),
      pltpu.SemaphoreType.DMA,
      pltpu.SemaphoreType.DMA,
      pltpu.SemaphoreType.DMA,
      pltpu.SemaphoreType.DMA,
      pltpu.SemaphoreType.DMA,
      pltpu.SemaphoreType.DMA,
  ]


def _a1_body(yf_hbm, eidx_hbm, out_hbm, idx_v, rows_v, acc_sh,
             g0, g1, g2, s0, s1, s2):
  def init(c, s, acc_sh):
    pltpu.sync_copy(yf_hbm.at[pl.ds(c * NPAD + s * TSLC, TSLC), :],
                    acc_sh.at[pl.ds(s * TSLC, TSLC), :])

  _agg_body(yf_hbm, eidx_hbm, out_hbm, idx_v, rows_v, acc_sh,
            (g0, g1, g2), (s0, s1, s2),
            nb=80, row0_fn=lambda c, s: s * 80,
            base_fn=lambda c: c * NPAD, init_fn=init)


def _a1_call(yf, eidx):
  return pl.kernel(
      _a1_body,
      out_type=jax.ShapeDtypeStruct((2, NPAD, 128), jnp.float32),
      mesh=_mesh(),
      scratch_types=_agg_scratch(),
  )(yf, eidx)


def _a2_body(y2_hbm, eidx_hbm, zero_hbm, out_hbm, idx_v, rows_v, acc_sh,
             g0, g1, g2, s0, s1, s2):
  def init(c, s, acc_sh):
    @pl.when(c == 0)
    def _():
      pltpu.sync_copy(y2_hbm.at[pl.ds(s * TSLC, TSLC), :],
                      acc_sh.at[pl.ds(s * TSLC, TSLC), :])

    @pl.when(c == 1)
    def _():
      pltpu.sync_copy(zero_hbm.at[pl.ds(s * TSLC, TSLC), :],
                      acc_sh.at[pl.ds(s * TSLC, TSLC), :])

  _agg_body(y2_hbm, eidx_hbm, out_hbm, idx_v, rows_v, acc_sh,
            (g0, g1, g2), (s0, s1, s2),
            nb=40, row0_fn=lambda c, s: (c * 16 + s) * 40,
            base_fn=lambda c: None, init_fn=init)


def _a2_call(y2, eidx, zeros):
  return pl.kernel(
      _a2_body,
      out_type=jax.ShapeDtypeStruct((2, NPAD, CLS), jnp.float32),
      mesh=_mesh(),
      scratch_types=[
          pltpu.VMEM((3, 2, K), jnp.int32),
          pltpu.VMEM((3, K, CLS), jnp.float32),
          pltpu.VMEM_SHARED((NPAD, CLS), jnp.float32),
          pltpu.SemaphoreType.DMA,
          pltpu.SemaphoreType.DMA,
          pltpu.SemaphoreType.DMA,
          pltpu.SemaphoreType.DMA,
          pltpu.SemaphoreType.DMA,
          pltpu.SemaphoreType.DMA,
      ],
      compiler_params=pltpu.CompilerParams(use_tc_tiling_on_sc=False),
  )(y2, eidx, zeros)


def _m0_body(x_ref, w_ref, xw_ref):
  xw_ref[...] = lax.dot_general(x_ref[...], w_ref[...], (((1,), (0,)), ((), ())),
                                precision=lax.Precision.HIGHEST,
                                preferred_element_type=jnp.float32)


def _m0_call(x, W1):
  return pl.pallas_call(
      _m0_body,
      grid=(RB,),
      in_specs=[
          pl.BlockSpec((BLK, F_IN), lambda i: (i, 0)),
          pl.BlockSpec((F_IN, HID), lambda i: (0, 0)),
      ],
      out_specs=pl.BlockSpec((BLK, HID), lambda i: (i, 0)),
      out_shape=jax.ShapeDtypeStruct((NPAD, HID), jnp.float32),
  )(x, W1)


def _m1_body(xw_ref, degT_ref, y_ref, dis_ref):
  deg = degT_ref[:, 0:1] + degT_ref[:, 1:2] + 1.0
  dis = lax.rsqrt(deg)
  y = xw_ref[...] * dis
  y_ref[0] = y[:, :128]
  y_ref[1] = y[:, 128:]
  dis_ref[...] = dis


def _m1_call(xw, degT):
  return pl.pallas_call(
      _m1_body,
      grid=(RB,),
      in_specs=[
          pl.BlockSpec((BLK, HID), lambda i: (i, 0)),
          pl.BlockSpec((BLK, 2), lambda i: (i, 0)),
      ],
      out_specs=[
          pl.BlockSpec((2, BLK, 128), lambda i: (0, i, 0)),
          pl.BlockSpec((BLK, 1), lambda i: (i, 0)),
      ],
      out_shape=[
          jax.ShapeDtypeStruct((2, NPAD, 128), jnp.float32),
          jax.ShapeDtypeStruct((NPAD, 1), jnp.float32),
      ],
  )(xw, degT)


def _m2_body(z_ref, dis_ref, b1_ref, w2_ref, y2_ref):
  cat = jnp.concatenate([z_ref[0], z_ref[1]], axis=1)
  dis = dis_ref[...]
  h = jnp.maximum(cat * dis + b1_ref[...], 0.0)
  y2 = lax.dot_general(h, w2_ref[...], (((1,), (0,)), ((), ())),
                       precision=lax.Precision.HIGHEST,
                       preferred_element_type=jnp.float32)
  y2_ref[...] = y2 * dis


def _m2_call(z1, dis, b1r, W2):
  return pl.pallas_call(
      _m2_body,
      grid=(RB,),
      in_specs=[
          pl.BlockSpec((2, BLK, 128), lambda i: (0, i, 0)),
          pl.BlockSpec((BLK, 1), lambda i: (i, 0)),
          pl.BlockSpec((1, HID), lambda i: (0, 0)),
          pl.BlockSpec((HID, CLS), lambda i: (0, 0)),
      ],
      out_specs=pl.BlockSpec((BLK, CLS), lambda i: (i, 0)),
      out_shape=jax.ShapeDtypeStruct((NPAD, CLS), jnp.float32),
  )(z1, dis, b1r, W2)


def _m3_body(p_ref, dis_ref, b2_ref, o_ref):
  z = p_ref[0] + p_ref[1]
  o = z * dis_ref[...] + b2_ref[...]
  m = jnp.max(o, axis=1, keepdims=True)
  e = jnp.exp(o - m)
  lse = jnp.log(jnp.sum(e, axis=1, keepdims=True)) + m
  o_ref[...] = o - lse


def _m3_call(parts, dis, b2r):
  return pl.pallas_call(
      _m3_body,
      grid=(RB,),
      in_specs=[
          pl.BlockSpec((2, BLK, CLS), lambda i: (0, i, 0)),
          pl.BlockSpec((BLK, 1), lambda i: (i, 0)),
          pl.BlockSpec((1, CLS), lambda i: (0, 0)),
      ],
      out_specs=pl.BlockSpec((BLK, CLS), lambda i: (i, 0)),
      out_shape=jax.ShapeDtypeStruct((NPAD, CLS), jnp.float32),
  )(parts, dis, b2r)


def kernel(x, edge_index, W1, b1, W2, b2):
  src = edge_index[0]
  dst = edge_index[1]
  padi = (jnp.arange(EPAD - E, dtype=jnp.int32) % (NPAD - N)) + N
  src2d = jnp.concatenate([src, padi]).reshape(ROWS, K)
  dst2d = jnp.concatenate([dst, padi]).reshape(ROWS, K)
  eidx = jnp.stack([src2d, dst2d], axis=1)

  degs = _deg_call(dst2d).reshape(2, NPAD)
  xw = _m0_call(x, W1)
  degT = degs.T
  y1, dis = _m1_call(xw, degT)
  z1 = _a1_call(y1.reshape(2 * NPAD, 128), eidx)
  y2 = _m2_call(z1, dis, b1.reshape(1, HID), W2)
  zeros = jnp.zeros((NPAD, CLS), jnp.float32)
  parts = _a2_call(y2, eidx, zeros)
  out = _m3_call(parts, dis, b2.reshape(1, CLS))
  return out[:N]


# speedup vs baseline: 20.0982x; 1.0467x over previous
import functools

import jax
import jax.numpy as jnp
from jax import lax
from jax.experimental import pallas as pl
from jax.experimental.pallas import tpu as pltpu
from jax.experimental.pallas import tpu_sc as plsc

N = 10000
E = 160000
F_IN = 256
HID = 256
CLS = 64

NPAD = 10112
TSLC = NPAD // 16
K = 128
EPAD = 32 * 40 * K
ROWS = EPAD // K
RB = 8
BLK = NPAD // RB


def _mesh():
  return plsc.VectorSubcoreMesh(
      core_axis_name="c", subcore_axis_name="s", num_cores=2, num_subcores=16)


def _deg_body(dst_hbm, out_hbm, idx_v, ones_v, zb_v, wout_v, hist_sh):
  c = lax.axis_index("c")
  s = lax.axis_index("s")
  wid = c * 16 + s
  ones16 = jnp.ones((16,), jnp.float32)
  zeros16 = jnp.zeros((16,), jnp.float32)
  for i in range(8):
    ones_v[pl.ds(i * 16, 16)] = ones16
    zb_v[pl.ds(i * 16, 16)] = zeros16
  for i in range(5):
    pltpu.sync_copy(zb_v, hist_sh.at[pl.ds(s * 640 + i * 128, 128)])
  plsc.subcore_barrier()

  @pl.loop(0, 40)
  def _(j):
    row = wid * 40 + j
    pltpu.sync_copy(dst_hbm.at[row], idx_v)
    pltpu.sync_copy(ones_v, hist_sh.at[idx_v], add=True)

  plsc.subcore_barrier()
  pltpu.sync_copy(hist_sh.at[pl.ds(s * TSLC, TSLC)], wout_v)
  pltpu.sync_copy(wout_v, out_hbm.at[pl.ds(c * NPAD + s * TSLC, TSLC)])


def _deg_call(dst2d):
  return pl.kernel(
      _deg_body,
      out_type=jax.ShapeDtypeStruct((2 * NPAD,), jnp.float32),
      mesh=_mesh(),
      scratch_types=[
          pltpu.VMEM((K,), jnp.int32),
          pltpu.VMEM((K,), jnp.float32),
          pltpu.VMEM((K,), jnp.float32),
          pltpu.VMEM((TSLC,), jnp.float32),
          pltpu.VMEM_SHARED((10240,), jnp.float32),
      ],
  )(dst2d)


def _agg_body(tab_hbm, eidx_hbm, out_hbm, idx_v, rows_v, acc_sh, gsems, ssems,
              *, nb, kk, rr, row0_fn, base_fn, init_fn):
  c = lax.axis_index("c")
  s = lax.axis_index("s")
  base = base_fn(c)
  row0 = row0_fn(c, s)

  init_fn(c, s, acc_sh)
  plsc.subcore_barrier()

  def fire_gather(j, b):
    pltpu.sync_copy(eidx_hbm.at[row0 + j], idx_v.at[b])
    if base is not None:
      for i in range(kk // 16):
        idx_v[b, 0, pl.ds(i * 16, 16)] = idx_v[b, 0, pl.ds(i * 16, 16)] + base
    pltpu.async_copy(tab_hbm.at[idx_v.at[b, 0]], rows_v.at[b], gsems[b])

  def wait_gather(b):
    pltpu.make_async_copy(tab_hbm.at[idx_v.at[b, 0]], rows_v.at[b],
                          gsems[b]).wait()

  def fire_scatter(b):
    pltpu.async_copy(rows_v.at[b], acc_sh.at[idx_v.at[b, 1]], ssems[b],
                     add=True)

  def wait_scatter(b):
    pltpu.make_async_copy(rows_v.at[b], acc_sh.at[idx_v.at[b, 1]],
                          ssems[b]).wait()

  for m in range(rr - 2):
    fire_gather(m, m)
  for j in (0, 1):
    fire_gather(rr - 2 + j, rr - 2 + j)
    wait_gather(j)
    fire_scatter(j)

  main_iters = ((nb - rr) // rr) * rr

  @pl.loop(2, 2 + main_iters, step=rr)
  def _(j0):
    for t in range(rr):
      b = (2 + t) % rr
      bg = t % rr
      wait_scatter(bg)
      fire_gather(j0 + t + rr - 2, bg)
      wait_gather(b)
      fire_scatter(b)

  for j in range(2 + main_iters, nb - rr + 2):
    wait_scatter((j - 2) % rr)
    fire_gather(j + rr - 2, (j - 2) % rr)
    wait_gather(j % rr)
    fire_scatter(j % rr)

  for j in range(nb - rr + 2, nb):
    wait_scatter((j - 2) % rr)
    wait_gather(j % rr)
    fire_scatter(j % rr)
  wait_scatter((nb - 2) % rr)
  wait_scatter((nb - 1) % rr)

  plsc.subcore_barrier()
  pltpu.sync_copy(acc_sh.at[pl.ds(s * TSLC, TSLC), :],
                  out_hbm.at[c, pl.ds(s * TSLC, TSLC), :])


def _agg_scratch(kk, rr, width):
  return [
      pltpu.VMEM((rr, 2, kk), jnp.int32),
      pltpu.VMEM((rr, kk, width), jnp.float32),
      pltpu.VMEM_SHARED((NPAD, width), jnp.float32),
  ] + [pltpu.SemaphoreType.DMA] * (2 * rr)


K1 = 64
NB1 = EPAD // 16 // K1


def _a1_body(yf_hbm, eidx_hbm, out_hbm, idx_v, rows_v, acc_sh,
             g0, g1, g2, g3, g4, s0, s1, s2, s3, s4):
  def init(c, s, acc_sh):
    pltpu.sync_copy(yf_hbm.at[pl.ds(c * NPAD + s * TSLC, TSLC), :],
                    acc_sh.at[pl.ds(s * TSLC, TSLC), :])

  _agg_body(yf_hbm, eidx_hbm, out_hbm, idx_v, rows_v, acc_sh,
            (g0, g1, g2, g3, g4), (s0, s1, s2, s3, s4),
            nb=NB1, kk=K1, rr=5, row0_fn=lambda c, s: s * NB1,
            base_fn=lambda c: c * NPAD, init_fn=init)


def _a1_call(yf, eidx):
  return pl.kernel(
      _a1_body,
      out_type=jax.ShapeDtypeStruct((2, NPAD, 128), jnp.float32),
      mesh=_mesh(),
      scratch_types=_agg_scratch(K1, 5, 128),
  )(yf, eidx)


def _a2_body(y2_hbm, eidx_hbm, zero_hbm, out_hbm, idx_v, rows_v, acc_sh,
             g0, g1, g2, g3, g4, g5, s0, s1, s2, s3, s4, s5):
  def init(c, s, acc_sh):
    @pl.when(c == 0)
    def _():
      pltpu.sync_copy(y2_hbm.at[pl.ds(s * TSLC, TSLC), :],
                      acc_sh.at[pl.ds(s * TSLC, TSLC), :])

    @pl.when(c == 1)
    def _():
      pltpu.sync_copy(zero_hbm.at[pl.ds(s * TSLC, TSLC), :],
                      acc_sh.at[pl.ds(s * TSLC, TSLC), :])

  _agg_body(y2_hbm, eidx_hbm, out_hbm, idx_v, rows_v, acc_sh,
            (g0, g1, g2, g3, g4, g5), (s0, s1, s2, s3, s4, s5),
            nb=40, kk=K, rr=6, row0_fn=lambda c, s: (c * 16 + s) * 40,
            base_fn=lambda c: None, init_fn=init)


def _a2_call(y2, eidx, zeros):
  return pl.kernel(
      _a2_body,
      out_type=jax.ShapeDtypeStruct((2, NPAD, CLS), jnp.float32),
      mesh=_mesh(),
      scratch_types=_agg_scratch(K, 6, CLS),
      compiler_params=pltpu.CompilerParams(use_tc_tiling_on_sc=False),
  )(y2, eidx, zeros)


def _m0_body(x_ref, w_ref, xw_ref):
  xw_ref[...] = lax.dot_general(x_ref[...], w_ref[...], (((1,), (0,)), ((), ())),
                                precision=lax.Precision.HIGHEST,
                                preferred_element_type=jnp.float32)


def _m0_call(x, W1):
  return pl.pallas_call(
      _m0_body,
      grid=(RB,),
      in_specs=[
          pl.BlockSpec((BLK, F_IN), lambda i: (i, 0)),
          pl.BlockSpec((F_IN, HID), lambda i: (0, 0)),
      ],
      out_specs=pl.BlockSpec((BLK, HID), lambda i: (i, 0)),
      out_shape=jax.ShapeDtypeStruct((NPAD, HID), jnp.float32),
  )(x, W1)


def _m1_body(xw_ref, degT_ref, y_ref, dis_ref):
  deg = degT_ref[:, 0:1] + degT_ref[:, 1:2] + 1.0
  dis = lax.rsqrt(deg)
  y = xw_ref[...] * dis
  y_ref[0] = y[:, :128]
  y_ref[1] = y[:, 128:]
  dis_ref[...] = dis


def _m1_call(xw, degT):
  return pl.pallas_call(
      _m1_body,
      grid=(RB,),
      in_specs=[
          pl.BlockSpec((BLK, HID), lambda i: (i, 0)),
          pl.BlockSpec((BLK, 2), lambda i: (i, 0)),
      ],
      out_specs=[
          pl.BlockSpec((2, BLK, 128), lambda i: (0, i, 0)),
          pl.BlockSpec((BLK, 1), lambda i: (i, 0)),
      ],
      out_shape=[
          jax.ShapeDtypeStruct((2, NPAD, 128), jnp.float32),
          jax.ShapeDtypeStruct((NPAD, 1), jnp.float32),
      ],
  )(xw, degT)


def _m2_body(z_ref, dis_ref, b1_ref, w2_ref, y2_ref):
  cat = jnp.concatenate([z_ref[0], z_ref[1]], axis=1)
  dis = dis_ref[...]
  h = jnp.maximum(cat * dis + b1_ref[...], 0.0)
  y2 = lax.dot_general(h, w2_ref[...], (((1,), (0,)), ((), ())),
                       precision=lax.Precision.HIGHEST,
                       preferred_element_type=jnp.float32)
  y2_ref[...] = y2 * dis


def _m2_call(z1, dis, b1r, W2):
  return pl.pallas_call(
      _m2_body,
      grid=(RB,),
      in_specs=[
          pl.BlockSpec((2, BLK, 128), lambda i: (0, i, 0)),
          pl.BlockSpec((BLK, 1), lambda i: (i, 0)),
          pl.BlockSpec((1, HID), lambda i: (0, 0)),
          pl.BlockSpec((HID, CLS), lambda i: (0, 0)),
      ],
      out_specs=pl.BlockSpec((BLK, CLS), lambda i: (i, 0)),
      out_shape=jax.ShapeDtypeStruct((NPAD, CLS), jnp.float32),
  )(z1, dis, b1r, W2)


def _m3_body(p_ref, dis_ref, b2_ref, o_ref):
  z = p_ref[0] + p_ref[1]
  o = z * dis_ref[...] + b2_ref[...]
  m = jnp.max(o, axis=1, keepdims=True)
  e = jnp.exp(o - m)
  lse = jnp.log(jnp.sum(e, axis=1, keepdims=True)) + m
  o_ref[...] = o - lse


def _m3_call(parts, dis, b2r):
  return pl.pallas_call(
      _m3_body,
      grid=(RB,),
      in_specs=[
          pl.BlockSpec((2, BLK, CLS), lambda i: (0, i, 0)),
          pl.BlockSpec((BLK, 1), lambda i: (i, 0)),
          pl.BlockSpec((1, CLS), lambda i: (0, 0)),
      ],
      out_specs=pl.BlockSpec((BLK, CLS), lambda i: (i, 0)),
      out_shape=jax.ShapeDtypeStruct((NPAD, CLS), jnp.float32),
  )(parts, dis, b2r)


def kernel(x, edge_index, W1, b1, W2, b2):
  src = edge_index[0]
  dst = edge_index[1]
  padi = (jnp.arange(EPAD - E, dtype=jnp.int32) % (NPAD - N)) + N
  src2d = jnp.concatenate([src, padi]).reshape(ROWS, K)
  dst2d = jnp.concatenate([dst, padi]).reshape(ROWS, K)
  eidx = jnp.stack([src2d, dst2d], axis=1)
  src2d1 = jnp.concatenate([src, padi]).reshape(EPAD // K1, K1)
  dst2d1 = jnp.concatenate([dst, padi]).reshape(EPAD // K1, K1)
  eidx1 = jnp.stack([src2d1, dst2d1], axis=1)

  degs = _deg_call(dst2d).reshape(2, NPAD)
  xw = _m0_call(x, W1)
  degT = degs.T
  y1, dis = _m1_call(xw, degT)
  z1 = _a1_call(y1.reshape(2 * NPAD, 128), eidx1)
  y2 = _m2_call(z1, dis, b1.reshape(1, HID), W2)
  zeros = jnp.zeros((NPAD, CLS), jnp.float32)
  parts = _a2_call(y2, eidx, zeros)
  out = _m3_call(parts, dis, b2.reshape(1, CLS))
  return out[:N]


# speedup vs baseline: 20.4604x; 1.0180x over previous
import functools

import jax
import jax.numpy as jnp
from jax import lax
from jax.experimental import pallas as pl
from jax.experimental.pallas import tpu as pltpu
from jax.experimental.pallas import tpu_sc as plsc

N = 10000
E = 160000
F_IN = 256
HID = 256
CLS = 64

NPAD = 10112
TSLC = NPAD // 16
K = 128
EPAD = 32 * 40 * K
ROWS = EPAD // K
RB = 8
BLK = NPAD // RB


def _mesh():
  return plsc.VectorSubcoreMesh(
      core_axis_name="c", subcore_axis_name="s", num_cores=2, num_subcores=16)


def _deg_body(dst_hbm, out_hbm, idx_v, ones_v, zb_v, wout_v, hist_sh):
  c = lax.axis_index("c")
  s = lax.axis_index("s")
  wid = c * 16 + s
  ones16 = jnp.ones((16,), jnp.float32)
  zeros16 = jnp.zeros((16,), jnp.float32)
  for i in range(8):
    ones_v[pl.ds(i * 16, 16)] = ones16
    zb_v[pl.ds(i * 16, 16)] = zeros16
  for i in range(5):
    pltpu.sync_copy(zb_v, hist_sh.at[pl.ds(s * 640 + i * 128, 128)])
  plsc.subcore_barrier()

  @pl.loop(0, 40)
  def _(j):
    row = wid * 40 + j
    pltpu.sync_copy(dst_hbm.at[row], idx_v)
    pltpu.sync_copy(ones_v, hist_sh.at[idx_v], add=True)

  plsc.subcore_barrier()
  pltpu.sync_copy(hist_sh.at[pl.ds(s * TSLC, TSLC)], wout_v)
  pltpu.sync_copy(wout_v, out_hbm.at[pl.ds(c * NPAD + s * TSLC, TSLC)])


def _deg_call(dst2d):
  return pl.kernel(
      _deg_body,
      out_type=jax.ShapeDtypeStruct((2 * NPAD,), jnp.float32),
      mesh=_mesh(),
      scratch_types=[
          pltpu.VMEM((K,), jnp.int32),
          pltpu.VMEM((K,), jnp.float32),
          pltpu.VMEM((K,), jnp.float32),
          pltpu.VMEM((TSLC,), jnp.float32),
          pltpu.VMEM_SHARED((10240,), jnp.float32),
      ],
  )(dst2d)


def _agg_body(tab_hbm, eidx_hbm, out_hbm, idx_v, rows_v, acc_sh, gsems, ssems,
              *, nb, kk, rr, row0_fn, base_fn, init_fn):
  c = lax.axis_index("c")
  s = lax.axis_index("s")
  base = base_fn(c)
  row0 = row0_fn(c, s)

  init_fn(c, s, acc_sh)
  plsc.subcore_barrier()

  def fire_gather(j, b):
    pltpu.sync_copy(eidx_hbm.at[row0 + j], idx_v.at[b])
    if base is not None:
      for i in range(kk // 16):
        idx_v[b, 0, pl.ds(i * 16, 16)] = idx_v[b, 0, pl.ds(i * 16, 16)] + base
    pltpu.async_copy(tab_hbm.at[idx_v.at[b, 0]], rows_v.at[b], gsems[b])

  def wait_gather(b):
    pltpu.make_async_copy(tab_hbm.at[idx_v.at[b, 0]], rows_v.at[b],
                          gsems[b]).wait()

  def fire_scatter(b):
    pltpu.async_copy(rows_v.at[b], acc_sh.at[idx_v.at[b, 1]], ssems[b],
                     add=True)

  def wait_scatter(b):
    pltpu.make_async_copy(rows_v.at[b], acc_sh.at[idx_v.at[b, 1]],
                          ssems[b]).wait()

  for m in range(rr - 2):
    fire_gather(m, m)
  for j in (0, 1):
    fire_gather(rr - 2 + j, rr - 2 + j)
    wait_gather(j)
    fire_scatter(j)

  main_iters = ((nb - rr) // rr) * rr

  @pl.loop(2, 2 + main_iters, step=rr)
  def _(j0):
    for t in range(rr):
      b = (2 + t) % rr
      bg = t % rr
      wait_scatter(bg)
      fire_gather(j0 + t + rr - 2, bg)
      wait_gather(b)
      fire_scatter(b)

  for j in range(2 + main_iters, nb - rr + 2):
    wait_scatter((j - 2) % rr)
    fire_gather(j + rr - 2, (j - 2) % rr)
    wait_gather(j % rr)
    fire_scatter(j % rr)

  for j in range(nb - rr + 2, nb):
    wait_scatter((j - 2) % rr)
    wait_gather(j % rr)
    fire_scatter(j % rr)
  wait_scatter((nb - 2) % rr)
  wait_scatter((nb - 1) % rr)

  plsc.subcore_barrier()
  pltpu.sync_copy(acc_sh.at[pl.ds(s * TSLC, TSLC), :],
                  out_hbm.at[c, pl.ds(s * TSLC, TSLC), :])


def _agg_scratch(kk, rr, width):
  return [
      pltpu.VMEM((rr, 2, kk), jnp.int32),
      pltpu.VMEM((rr, kk, width), jnp.float32),
      pltpu.VMEM_SHARED((NPAD, width), jnp.float32),
  ] + [pltpu.SemaphoreType.DMA] * (2 * rr)


K1 = 64
NB1 = EPAD // 16 // K1


def _a1_body(yf_hbm, eidx_hbm, out_hbm, idx_v, rows_v, acc_sh,
             g0, g1, g2, g3, g4, s0, s1, s2, s3, s4):
  def init(c, s, acc_sh):
    pltpu.sync_copy(yf_hbm.at[pl.ds(c * NPAD + s * TSLC, TSLC), :],
                    acc_sh.at[pl.ds(s * TSLC, TSLC), :])

  _agg_body(yf_hbm, eidx_hbm, out_hbm, idx_v, rows_v, acc_sh,
            (g0, g1, g2, g3, g4), (s0, s1, s2, s3, s4),
            nb=NB1, kk=K1, rr=5, row0_fn=lambda c, s: s * NB1,
            base_fn=lambda c: c * NPAD, init_fn=init)


def _a1_call(yf, eidx):
  return pl.kernel(
      _a1_body,
      out_type=jax.ShapeDtypeStruct((2, NPAD, 128), jnp.float32),
      mesh=_mesh(),
      scratch_types=_agg_scratch(K1, 5, 128),
  )(yf, eidx)


def _a2_body(y2_hbm, eidx_hbm, zero_hbm, out_hbm, idx_v, rows_v, acc_sh,
             g0, g1, g2, g3, g4, g5, s0, s1, s2, s3, s4, s5):
  def init(c, s, acc_sh):
    @pl.when(c == 0)
    def _():
      pltpu.sync_copy(y2_hbm.at[pl.ds(s * TSLC, TSLC), :],
                      acc_sh.at[pl.ds(s * TSLC, TSLC), :])

    @pl.when(c == 1)
    def _():
      pltpu.sync_copy(zero_hbm.at[pl.ds(s * TSLC, TSLC), :],
                      acc_sh.at[pl.ds(s * TSLC, TSLC), :])

  _agg_body(y2_hbm, eidx_hbm, out_hbm, idx_v, rows_v, acc_sh,
            (g0, g1, g2, g3, g4, g5), (s0, s1, s2, s3, s4, s5),
            nb=40, kk=K, rr=6, row0_fn=lambda c, s: (c * 16 + s) * 40,
            base_fn=lambda c: None, init_fn=init)


def _a2_call(y2, eidx, zeros):
  return pl.kernel(
      _a2_body,
      out_type=jax.ShapeDtypeStruct((2, NPAD, CLS), jnp.float32),
      mesh=_mesh(),
      scratch_types=_agg_scratch(K, 6, CLS),
      compiler_params=pltpu.CompilerParams(use_tc_tiling_on_sc=False),
  )(y2, eidx, zeros)


def _m0_body(x_ref, w_ref, xw_ref):
  xw_ref[...] = lax.dot_general(x_ref[...], w_ref[...], (((1,), (0,)), ((), ())),
                                precision=lax.Precision.HIGHEST,
                                preferred_element_type=jnp.float32)


def _m0_call(x, W1):
  return pl.pallas_call(
      _m0_body,
      grid=(RB,),
      in_specs=[
          pl.BlockSpec((BLK, F_IN), lambda i: (i, 0)),
          pl.BlockSpec((F_IN, HID), lambda i: (0, 0)),
      ],
      out_specs=pl.BlockSpec((BLK, HID), lambda i: (i, 0)),
      out_shape=jax.ShapeDtypeStruct((NPAD, HID), jnp.float32),
  )(x, W1)


def _m1_body(xw_ref, degT_ref, y_ref, dis_ref):
  deg = degT_ref[:, 0:1] + degT_ref[:, 1:2] + 1.0
  dis = lax.rsqrt(deg)
  y = xw_ref[...] * dis
  y_ref[0] = y[:, :128]
  y_ref[1] = y[:, 128:]
  dis_ref[...] = dis


def _m1_call(xw, degT):
  return pl.pallas_call(
      _m1_body,
      grid=(RB,),
      in_specs=[
          pl.BlockSpec((BLK, HID), lambda i: (i, 0)),
          pl.BlockSpec((BLK, 2), lambda i: (i, 0)),
      ],
      out_specs=[
          pl.BlockSpec((2, BLK, 128), lambda i: (0, i, 0)),
          pl.BlockSpec((BLK, 1), lambda i: (i, 0)),
      ],
      out_shape=[
          jax.ShapeDtypeStruct((2, NPAD, 128), jnp.float32),
          jax.ShapeDtypeStruct((NPAD, 1), jnp.float32),
      ],
  )(xw, degT)


def _m2_body(z_ref, dis_ref, b1_ref, w2_ref, y2_ref):
  cat = jnp.concatenate([z_ref[0], z_ref[1]], axis=1)
  dis = dis_ref[...]
  h = jnp.maximum(cat * dis + b1_ref[...], 0.0)
  y2 = lax.dot_general(h, w2_ref[...], (((1,), (0,)), ((), ())),
                       precision=lax.Precision.HIGHEST,
                       preferred_element_type=jnp.float32)
  y2_ref[...] = y2 * dis


def _m2_call(z1, dis, b1r, W2):
  return pl.pallas_call(
      _m2_body,
      grid=(RB,),
      in_specs=[
          pl.BlockSpec((2, BLK, 128), lambda i: (0, i, 0)),
          pl.BlockSpec((BLK, 1), lambda i: (i, 0)),
          pl.BlockSpec((1, HID), lambda i: (0, 0)),
          pl.BlockSpec((HID, CLS), lambda i: (0, 0)),
      ],
      out_specs=pl.BlockSpec((BLK, CLS), lambda i: (i, 0)),
      out_shape=jax.ShapeDtypeStruct((NPAD, CLS), jnp.float32),
  )(z1, dis, b1r, W2)


def _m3_body(p_ref, dis_ref, b2_ref, o_ref):
  z = p_ref[0] + p_ref[1]
  o = z * dis_ref[...] + b2_ref[...]
  m = jnp.max(o, axis=1, keepdims=True)
  e = jnp.exp(o - m)
  lse = jnp.log(jnp.sum(e, axis=1, keepdims=True)) + m
  o_ref[...] = o - lse


def _m3_call(parts, dis, b2r):
  return pl.pallas_call(
      _m3_body,
      grid=(RB,),
      in_specs=[
          pl.BlockSpec((2, BLK, CLS), lambda i: (0, i, 0)),
          pl.BlockSpec((BLK, 1), lambda i: (i, 0)),
          pl.BlockSpec((1, CLS), lambda i: (0, 0)),
      ],
      out_specs=pl.BlockSpec((BLK, CLS), lambda i: (i, 0)),
      out_shape=jax.ShapeDtypeStruct((N, CLS), jnp.float32),
  )(parts, dis, b2r)


def kernel(x, edge_index, W1, b1, W2, b2):
  src = edge_index[0]
  dst = edge_index[1]
  padi = (jnp.arange(EPAD - E, dtype=jnp.int32) % (NPAD - N)) + N
  src2d = jnp.concatenate([src, padi]).reshape(ROWS, K)
  dst2d = jnp.concatenate([dst, padi]).reshape(ROWS, K)
  eidx = jnp.stack([src2d, dst2d], axis=1)
  src2d1 = jnp.concatenate([src, padi]).reshape(EPAD // K1, K1)
  dst2d1 = jnp.concatenate([dst, padi]).reshape(EPAD // K1, K1)
  eidx1 = jnp.stack([src2d1, dst2d1], axis=1)

  degs = _deg_call(dst2d).reshape(2, NPAD)
  xw = _m0_call(x, W1)
  degT = degs.T
  y1, dis = _m1_call(xw, degT)
  z1 = _a1_call(y1.reshape(2 * NPAD, 128), eidx1)
  y2 = _m2_call(z1, dis, b1.reshape(1, HID), W2)
  zeros = jnp.zeros((NPAD, CLS), jnp.float32)
  parts = _a2_call(y2, eidx, zeros)
  return _m3_call(parts, dis, b2.reshape(1, CLS))


# speedup vs baseline: 20.5011x; 1.0020x over previous
import functools

import jax
import jax.numpy as jnp
from jax import lax
from jax.experimental import pallas as pl
from jax.experimental.pallas import tpu as pltpu
from jax.experimental.pallas import tpu_sc as plsc

N = 10000
E = 160000
F_IN = 256
HID = 256
CLS = 64

NPAD = 10112
TSLC = NPAD // 16
K = 128
EPAD = 32 * 40 * K
ROWS = EPAD // K
RB = 8
BLK = NPAD // RB


def _mesh():
  return plsc.VectorSubcoreMesh(
      core_axis_name="c", subcore_axis_name="s", num_cores=2, num_subcores=16)


def _deg_body(dst_hbm, out_hbm, idx_v, ones_v, zb_v, wout_v, hist_sh):
  c = lax.axis_index("c")
  s = lax.axis_index("s")
  wid = c * 16 + s
  ones16 = jnp.ones((16,), jnp.float32)
  zeros16 = jnp.zeros((16,), jnp.float32)
  for i in range(8):
    ones_v[pl.ds(i * 16, 16)] = ones16
    zb_v[pl.ds(i * 16, 16)] = zeros16
  for i in range(5):
    pltpu.sync_copy(zb_v, hist_sh.at[pl.ds(s * 640 + i * 128, 128)])
  plsc.subcore_barrier()

  @pl.loop(0, 40)
  def _(j):
    row = wid * 40 + j
    pltpu.sync_copy(dst_hbm.at[row], idx_v)
    pltpu.sync_copy(ones_v, hist_sh.at[idx_v], add=True)

  plsc.subcore_barrier()
  pltpu.sync_copy(hist_sh.at[pl.ds(s * TSLC, TSLC)], wout_v)
  pltpu.sync_copy(wout_v, out_hbm.at[pl.ds(c * NPAD + s * TSLC, TSLC)])


def _deg_call(dst2d):
  return pl.kernel(
      _deg_body,
      out_type=jax.ShapeDtypeStruct((2 * NPAD,), jnp.float32),
      mesh=_mesh(),
      scratch_types=[
          pltpu.VMEM((K,), jnp.int32),
          pltpu.VMEM((K,), jnp.float32),
          pltpu.VMEM((K,), jnp.float32),
          pltpu.VMEM((TSLC,), jnp.float32),
          pltpu.VMEM_SHARED((10240,), jnp.float32),
      ],
  )(dst2d)


def _agg_body(tab_hbm, eidx_hbm, out_hbm, idx_v, rows_v, acc_sh, gsems, ssems,
              *, nb, kk, rr, row0_fn, base_fn, init_fn):
  c = lax.axis_index("c")
  s = lax.axis_index("s")
  base = base_fn(c)
  row0 = row0_fn(c, s)

  def fire_gather(j, b):
    pltpu.sync_copy(eidx_hbm.at[row0 + j], idx_v.at[b])
    if base is not None:
      for i in range(kk // 16):
        idx_v[b, 0, pl.ds(i * 16, 16)] = idx_v[b, 0, pl.ds(i * 16, 16)] + base
    pltpu.async_copy(tab_hbm.at[idx_v.at[b, 0]], rows_v.at[b], gsems[b])

  def wait_gather(b):
    pltpu.make_async_copy(tab_hbm.at[idx_v.at[b, 0]], rows_v.at[b],
                          gsems[b]).wait()

  def fire_scatter(b):
    pltpu.async_copy(rows_v.at[b], acc_sh.at[idx_v.at[b, 1]], ssems[b],
                     add=True)

  def wait_scatter(b):
    pltpu.make_async_copy(rows_v.at[b], acc_sh.at[idx_v.at[b, 1]],
                          ssems[b]).wait()

  for m in range(rr - 2):
    fire_gather(m, m)
  init_fn(c, s, acc_sh)
  plsc.subcore_barrier()
  for j in (0, 1):
    fire_gather(rr - 2 + j, rr - 2 + j)
    wait_gather(j)
    fire_scatter(j)

  main_iters = ((nb - rr) // rr) * rr

  @pl.loop(2, 2 + main_iters, step=rr)
  def _(j0):
    for t in range(rr):
      b = (2 + t) % rr
      bg = t % rr
      wait_scatter(bg)
      fire_gather(j0 + t + rr - 2, bg)
      wait_gather(b)
      fire_scatter(b)

  for j in range(2 + main_iters, nb - rr + 2):
    wait_scatter((j - 2) % rr)
    fire_gather(j + rr - 2, (j - 2) % rr)
    wait_gather(j % rr)
    fire_scatter(j % rr)

  for j in range(nb - rr + 2, nb):
    wait_scatter((j - 2) % rr)
    wait_gather(j % rr)
    fire_scatter(j % rr)
  wait_scatter((nb - 2) % rr)
  wait_scatter((nb - 1) % rr)

  plsc.subcore_barrier()
  pltpu.sync_copy(acc_sh.at[pl.ds(s * TSLC, TSLC), :],
                  out_hbm.at[c, pl.ds(s * TSLC, TSLC), :])


def _agg_scratch(kk, rr, width):
  return [
      pltpu.VMEM((rr, 2, kk), jnp.int32),
      pltpu.VMEM((rr, kk, width), jnp.float32),
      pltpu.VMEM_SHARED((NPAD, width), jnp.float32),
  ] + [pltpu.SemaphoreType.DMA] * (2 * rr)


K1 = 64
NB1 = EPAD // 16 // K1


def _a1_body(yf_hbm, eidx_hbm, out_hbm, idx_v, rows_v, acc_sh,
             g0, g1, g2, g3, g4, s0, s1, s2, s3, s4):
  def init(c, s, acc_sh):
    pltpu.sync_copy(yf_hbm.at[pl.ds(c * NPAD + s * TSLC, TSLC), :],
                    acc_sh.at[pl.ds(s * TSLC, TSLC), :])

  _agg_body(yf_hbm, eidx_hbm, out_hbm, idx_v, rows_v, acc_sh,
            (g0, g1, g2, g3, g4), (s0, s1, s2, s3, s4),
            nb=NB1, kk=K1, rr=5, row0_fn=lambda c, s: s * NB1,
            base_fn=lambda c: c * NPAD, init_fn=init)


def _a1_call(yf, eidx):
  return pl.kernel(
      _a1_body,
      out_type=jax.ShapeDtypeStruct((2, NPAD, 128), jnp.float32),
      mesh=_mesh(),
      scratch_types=_agg_scratch(K1, 5, 128),
  )(yf, eidx)


def _a2_body(y2_hbm, eidx_hbm, zero_hbm, out_hbm, idx_v, rows_v, acc_sh,
             g0, g1, g2, g3, g4, g5, s0, s1, s2, s3, s4, s5):
  def init(c, s, acc_sh):
    @pl.when(c == 0)
    def _():
      pltpu.sync_copy(y2_hbm.at[pl.ds(s * TSLC, TSLC), :],
                      acc_sh.at[pl.ds(s * TSLC, TSLC), :])

    @pl.when(c == 1)
    def _():
      pltpu.sync_copy(zero_hbm.at[pl.ds(s * TSLC, TSLC), :],
                      acc_sh.at[pl.ds(s * TSLC, TSLC), :])

  _agg_body(y2_hbm, eidx_hbm, out_hbm, idx_v, rows_v, acc_sh,
            (g0, g1, g2, g3, g4, g5), (s0, s1, s2, s3, s4, s5),
            nb=40, kk=K, rr=6, row0_fn=lambda c, s: (c * 16 + s) * 40,
            base_fn=lambda c: None, init_fn=init)


def _a2_call(y2, eidx, zeros):
  return pl.kernel(
      _a2_body,
      out_type=jax.ShapeDtypeStruct((2, NPAD, CLS), jnp.float32),
      mesh=_mesh(),
      scratch_types=_agg_scratch(K, 6, CLS),
      compiler_params=pltpu.CompilerParams(use_tc_tiling_on_sc=False),
  )(y2, eidx, zeros)


def _m0_body(x_ref, w_ref, xw_ref):
  xw_ref[...] = lax.dot_general(x_ref[...], w_ref[...], (((1,), (0,)), ((), ())),
                                precision=lax.Precision.HIGHEST,
                                preferred_element_type=jnp.float32)


def _m0_call(x, W1):
  return pl.pallas_call(
      _m0_body,
      grid=(RB,),
      in_specs=[
          pl.BlockSpec((BLK, F_IN), lambda i: (i, 0)),
          pl.BlockSpec((F_IN, HID), lambda i: (0, 0)),
      ],
      out_specs=pl.BlockSpec((BLK, HID), lambda i: (i, 0)),
      out_shape=jax.ShapeDtypeStruct((NPAD, HID), jnp.float32),
  )(x, W1)


def _m1_body(xw_ref, degT_ref, y_ref, dis_ref):
  deg = degT_ref[:, 0:1] + degT_ref[:, 1:2] + 1.0
  dis = lax.rsqrt(deg)
  y = xw_ref[...] * dis
  y_ref[0] = y[:, :128]
  y_ref[1] = y[:, 128:]
  dis_ref[...] = dis


def _m1_call(xw, degT):
  return pl.pallas_call(
      _m1_body,
      grid=(RB,),
      in_specs=[
          pl.BlockSpec((BLK, HID), lambda i: (i, 0)),
          pl.BlockSpec((BLK, 2), lambda i: (i, 0)),
      ],
      out_specs=[
          pl.BlockSpec((2, BLK, 128), lambda i: (0, i, 0)),
          pl.BlockSpec((BLK, 1), lambda i: (i, 0)),
      ],
      out_shape=[
          jax.ShapeDtypeStruct((2, NPAD, 128), jnp.float32),
          jax.ShapeDtypeStruct((NPAD, 1), jnp.float32),
      ],
  )(xw, degT)


def _m2_body(z_ref, dis_ref, b1_ref, w2_ref, y2_ref):
  cat = jnp.concatenate([z_ref[0], z_ref[1]], axis=1)
  dis = dis_ref[...]
  h = jnp.maximum(cat * dis + b1_ref[...], 0.0)
  y2 = lax.dot_general(h, w2_ref[...], (((1,), (0,)), ((), ())),
                       precision=lax.Precision.HIGHEST,
                       preferred_element_type=jnp.float32)
  y2_ref[...] = y2 * dis


def _m2_call(z1, dis, b1r, W2):
  return pl.pallas_call(
      _m2_body,
      grid=(RB,),
      in_specs=[
          pl.BlockSpec((2, BLK, 128), lambda i: (0, i, 0)),
          pl.BlockSpec((BLK, 1), lambda i: (i, 0)),
          pl.BlockSpec((1, HID), lambda i: (0, 0)),
          pl.BlockSpec((HID, CLS), lambda i: (0, 0)),
      ],
      out_specs=pl.BlockSpec((BLK, CLS), lambda i: (i, 0)),
      out_shape=jax.ShapeDtypeStruct((NPAD, CLS), jnp.float32),
  )(z1, dis, b1r, W2)


def _m3_body(p_ref, dis_ref, b2_ref, o_ref):
  z = p_ref[0] + p_ref[1]
  o = z * dis_ref[...] + b2_ref[...]
  m = jnp.max(o, axis=1, keepdims=True)
  e = jnp.exp(o - m)
  lse = jnp.log(jnp.sum(e, axis=1, keepdims=True)) + m
  o_ref[...] = o - lse


def _m3_call(parts, dis, b2r):
  return pl.pallas_call(
      _m3_body,
      grid=(RB,),
      in_specs=[
          pl.BlockSpec((2, BLK, CLS), lambda i: (0, i, 0)),
          pl.BlockSpec((BLK, 1), lambda i: (i, 0)),
          pl.BlockSpec((1, CLS), lambda i: (0, 0)),
      ],
      out_specs=pl.BlockSpec((BLK, CLS), lambda i: (i, 0)),
      out_shape=jax.ShapeDtypeStruct((N, CLS), jnp.float32),
  )(parts, dis, b2r)


def kernel(x, edge_index, W1, b1, W2, b2):
  src = edge_index[0]
  dst = edge_index[1]
  padi = (jnp.arange(EPAD - E, dtype=jnp.int32) % (NPAD - N)) + N
  src2d = jnp.concatenate([src, padi]).reshape(ROWS, K)
  dst2d = jnp.concatenate([dst, padi]).reshape(ROWS, K)
  eidx = jnp.stack([src2d, dst2d], axis=1)
  src2d1 = jnp.concatenate([src, padi]).reshape(EPAD // K1, K1)
  dst2d1 = jnp.concatenate([dst, padi]).reshape(EPAD // K1, K1)
  eidx1 = jnp.stack([src2d1, dst2d1], axis=1)

  degs = _deg_call(dst2d).reshape(2, NPAD)
  xw = _m0_call(x, W1)
  degT = degs.T
  y1, dis = _m1_call(xw, degT)
  z1 = _a1_call(y1.reshape(2 * NPAD, 128), eidx1)
  y2 = _m2_call(z1, dis, b1.reshape(1, HID), W2)
  zeros = jnp.zeros((NPAD, CLS), jnp.float32)
  parts = _a2_call(y2, eidx, zeros)
  return _m3_call(parts, dis, b2.reshape(1, CLS))


# speedup vs baseline: 20.8050x; 1.0148x over previous
import functools

import jax
import jax.numpy as jnp
from jax import lax
from jax.experimental import pallas as pl
from jax.experimental.pallas import tpu as pltpu
from jax.experimental.pallas import tpu_sc as plsc

N = 10000
E = 160000
F_IN = 256
HID = 256
CLS = 64

NPAD = 10112
TSLC = NPAD // 16
K = 128
EPAD = 32 * 40 * K
ROWS = EPAD // K
RB = 8
BLK = NPAD // RB


def _mesh():
  return plsc.VectorSubcoreMesh(
      core_axis_name="c", subcore_axis_name="s", num_cores=2, num_subcores=16)


def _deg_body(dst_hbm, out_hbm, idx_v, ones_v, zb_v, wout_v, hist_sh):
  c = lax.axis_index("c")
  s = lax.axis_index("s")
  wid = c * 16 + s
  ones16 = jnp.ones((16,), jnp.float32)
  zeros16 = jnp.zeros((16,), jnp.float32)
  for i in range(8):
    ones_v[pl.ds(i * 16, 16)] = ones16
    zb_v[pl.ds(i * 16, 16)] = zeros16
  for i in range(5):
    pltpu.sync_copy(zb_v, hist_sh.at[pl.ds(s * 640 + i * 128, 128)])
  plsc.subcore_barrier()

  @pl.loop(0, 40)
  def _(j):
    row = wid * 40 + j
    pltpu.sync_copy(dst_hbm.at[row], idx_v)
    pltpu.sync_copy(ones_v, hist_sh.at[idx_v], add=True)

  plsc.subcore_barrier()
  pltpu.sync_copy(hist_sh.at[pl.ds(s * TSLC, TSLC)], wout_v)
  pltpu.sync_copy(wout_v, out_hbm.at[pl.ds(c * NPAD + s * TSLC, TSLC)])


def _deg_call(dst2d):
  return pl.kernel(
      _deg_body,
      out_type=jax.ShapeDtypeStruct((2 * NPAD,), jnp.float32),
      mesh=_mesh(),
      scratch_types=[
          pltpu.VMEM((K,), jnp.int32),
          pltpu.VMEM((K,), jnp.float32),
          pltpu.VMEM((K,), jnp.float32),
          pltpu.VMEM((TSLC,), jnp.float32),
          pltpu.VMEM_SHARED((10240,), jnp.float32),
      ],
  )(dst2d)


def _agg_body(tab_hbm, eidx_hbm, out_hbm, idx_v, rows_v, acc_sh, gsems, ssems,
              *, nb, kk, rr, row0_fn, base_fn, init_fn):
  c = lax.axis_index("c")
  s = lax.axis_index("s")
  base = base_fn(c)
  row0 = row0_fn(c, s)

  def fire_gather(j, b):
    pltpu.sync_copy(eidx_hbm.at[row0 + j], idx_v.at[b])
    if base is not None:
      for i in range(kk // 16):
        idx_v[b, 0, pl.ds(i * 16, 16)] = idx_v[b, 0, pl.ds(i * 16, 16)] + base
    pltpu.async_copy(tab_hbm.at[idx_v.at[b, 0]], rows_v.at[b], gsems[b])

  def wait_gather(b):
    pltpu.make_async_copy(tab_hbm.at[idx_v.at[b, 0]], rows_v.at[b],
                          gsems[b]).wait()

  def fire_scatter(b):
    pltpu.async_copy(rows_v.at[b], acc_sh.at[idx_v.at[b, 1]], ssems[b],
                     add=True)

  def wait_scatter(b):
    pltpu.make_async_copy(rows_v.at[b], acc_sh.at[idx_v.at[b, 1]],
                          ssems[b]).wait()

  for m in range(rr - 2):
    fire_gather(m, m)
  init_fn(c, s, acc_sh)
  plsc.subcore_barrier()
  for j in (0, 1):
    fire_gather(rr - 2 + j, rr - 2 + j)
    wait_gather(j)
    fire_scatter(j)

  main_iters = ((nb - rr) // rr) * rr

  @pl.loop(2, 2 + main_iters, step=rr)
  def _(j0):
    for t in range(rr):
      b = (2 + t) % rr
      bg = t % rr
      wait_scatter(bg)
      fire_gather(j0 + t + rr - 2, bg)
      wait_gather(b)
      fire_scatter(b)

  for j in range(2 + main_iters, nb - rr + 2):
    wait_scatter((j - 2) % rr)
    fire_gather(j + rr - 2, (j - 2) % rr)
    wait_gather(j % rr)
    fire_scatter(j % rr)

  for j in range(nb - rr + 2, nb):
    wait_scatter((j - 2) % rr)
    wait_gather(j % rr)
    fire_scatter(j % rr)
  wait_scatter((nb - 2) % rr)
  wait_scatter((nb - 1) % rr)

  plsc.subcore_barrier()
  pltpu.sync_copy(acc_sh.at[pl.ds(s * TSLC, TSLC), :],
                  out_hbm.at[c, pl.ds(s * TSLC, TSLC), :])


def _agg_scratch(kk, rr, width):
  return [
      pltpu.VMEM((rr, 2, kk), jnp.int32),
      pltpu.VMEM((rr, kk, width), jnp.float32),
      pltpu.VMEM_SHARED((NPAD, width), jnp.float32),
  ] + [pltpu.SemaphoreType.DMA] * (2 * rr)


K1 = 64
NB1 = EPAD // 16 // K1


def _a1_body(yf_hbm, eidx_hbm, out_hbm, idx_v, rows_v, acc_sh,
             g0, g1, g2, g3, g4, s0, s1, s2, s3, s4):
  def init(c, s, acc_sh):
    pltpu.sync_copy(yf_hbm.at[pl.ds(c * NPAD + s * TSLC, TSLC), :],
                    acc_sh.at[pl.ds(s * TSLC, TSLC), :])

  _agg_body(yf_hbm, eidx_hbm, out_hbm, idx_v, rows_v, acc_sh,
            (g0, g1, g2, g3, g4), (s0, s1, s2, s3, s4),
            nb=NB1, kk=K1, rr=5, row0_fn=lambda c, s: s * NB1,
            base_fn=lambda c: c * NPAD, init_fn=init)


def _a1_call(yf, eidx):
  return pl.kernel(
      _a1_body,
      out_type=jax.ShapeDtypeStruct((2, NPAD, 128), jnp.float32),
      mesh=_mesh(),
      scratch_types=_agg_scratch(K1, 5, 128),
  )(yf, eidx)


def _a2_body(y2_hbm, eidx_hbm, zero_hbm, out_hbm, idx_v, rows_v, acc_sh,
             g0, g1, g2, g3, g4, g5, s0, s1, s2, s3, s4, s5):
  def init(c, s, acc_sh):
    @pl.when(c == 0)
    def _():
      pltpu.sync_copy(y2_hbm.at[pl.ds(s * TSLC, TSLC), :],
                      acc_sh.at[pl.ds(s * TSLC, TSLC), :])

    @pl.when(c == 1)
    def _():
      pltpu.sync_copy(zero_hbm.at[pl.ds(s * TSLC, TSLC), :],
                      acc_sh.at[pl.ds(s * TSLC, TSLC), :])

  _agg_body(y2_hbm, eidx_hbm, out_hbm, idx_v, rows_v, acc_sh,
            (g0, g1, g2, g3, g4, g5), (s0, s1, s2, s3, s4, s5),
            nb=40, kk=K, rr=6, row0_fn=lambda c, s: (c * 16 + s) * 40,
            base_fn=lambda c: None, init_fn=init)


def _a2_call(y2, eidx, zeros):
  return pl.kernel(
      _a2_body,
      out_type=jax.ShapeDtypeStruct((2, NPAD, CLS), jnp.float32),
      mesh=_mesh(),
      scratch_types=_agg_scratch(K, 6, CLS),
      compiler_params=pltpu.CompilerParams(use_tc_tiling_on_sc=False),
  )(y2, eidx, zeros)


def _m0_body(x_ref, w_ref, xw_ref):
  xw_ref[...] = lax.dot_general(x_ref[...], w_ref[...], (((1,), (0,)), ((), ())),
                                preferred_element_type=jnp.float32)


def _m0_call(x, W1):
  return pl.pallas_call(
      _m0_body,
      grid=(RB,),
      in_specs=[
          pl.BlockSpec((BLK, F_IN), lambda i: (i, 0)),
          pl.BlockSpec((F_IN, HID), lambda i: (0, 0)),
      ],
      out_specs=pl.BlockSpec((BLK, HID), lambda i: (i, 0)),
      out_shape=jax.ShapeDtypeStruct((NPAD, HID), jnp.float32),
  )(x, W1)


def _m1_body(xw_ref, degT_ref, y_ref, dis_ref):
  deg = degT_ref[:, 0:1] + degT_ref[:, 1:2] + 1.0
  dis = lax.rsqrt(deg)
  y = xw_ref[...] * dis
  y_ref[0] = y[:, :128]
  y_ref[1] = y[:, 128:]
  dis_ref[...] = dis


def _m1_call(xw, degT):
  return pl.pallas_call(
      _m1_body,
      grid=(RB,),
      in_specs=[
          pl.BlockSpec((BLK, HID), lambda i: (i, 0)),
          pl.BlockSpec((BLK, 2), lambda i: (i, 0)),
      ],
      out_specs=[
          pl.BlockSpec((2, BLK, 128), lambda i: (0, i, 0)),
          pl.BlockSpec((BLK, 1), lambda i: (i, 0)),
      ],
      out_shape=[
          jax.ShapeDtypeStruct((2, NPAD, 128), jnp.float32),
          jax.ShapeDtypeStruct((NPAD, 1), jnp.float32),
      ],
  )(xw, degT)


def _m2_body(z_ref, dis_ref, b1_ref, w2_ref, y2_ref):
  cat = jnp.concatenate([z_ref[0], z_ref[1]], axis=1)
  dis = dis_ref[...]
  h = jnp.maximum(cat * dis + b1_ref[...], 0.0)
  y2 = lax.dot_general(h, w2_ref[...], (((1,), (0,)), ((), ())),
                       preferred_element_type=jnp.float32)
  y2_ref[...] = y2 * dis


def _m2_call(z1, dis, b1r, W2):
  return pl.pallas_call(
      _m2_body,
      grid=(RB,),
      in_specs=[
          pl.BlockSpec((2, BLK, 128), lambda i: (0, i, 0)),
          pl.BlockSpec((BLK, 1), lambda i: (i, 0)),
          pl.BlockSpec((1, HID), lambda i: (0, 0)),
          pl.BlockSpec((HID, CLS), lambda i: (0, 0)),
      ],
      out_specs=pl.BlockSpec((BLK, CLS), lambda i: (i, 0)),
      out_shape=jax.ShapeDtypeStruct((NPAD, CLS), jnp.float32),
  )(z1, dis, b1r, W2)


def _m3_body(p_ref, dis_ref, b2_ref, o_ref):
  z = p_ref[0] + p_ref[1]
  o = z * dis_ref[...] + b2_ref[...]
  m = jnp.max(o, axis=1, keepdims=True)
  e = jnp.exp(o - m)
  lse = jnp.log(jnp.sum(e, axis=1, keepdims=True)) + m
  o_ref[...] = o - lse


def _m3_call(parts, dis, b2r):
  return pl.pallas_call(
      _m3_body,
      grid=(RB,),
      in_specs=[
          pl.BlockSpec((2, BLK, CLS), lambda i: (0, i, 0)),
          pl.BlockSpec((BLK, 1), lambda i: (i, 0)),
          pl.BlockSpec((1, CLS), lambda i: (0, 0)),
      ],
      out_specs=pl.BlockSpec((BLK, CLS), lambda i: (i, 0)),
      out_shape=jax.ShapeDtypeStruct((N, CLS), jnp.float32),
  )(parts, dis, b2r)


def kernel(x, edge_index, W1, b1, W2, b2):
  src = edge_index[0]
  dst = edge_index[1]
  padi = (jnp.arange(EPAD - E, dtype=jnp.int32) % (NPAD - N)) + N
  src2d = jnp.concatenate([src, padi]).reshape(ROWS, K)
  dst2d = jnp.concatenate([dst, padi]).reshape(ROWS, K)
  eidx = jnp.stack([src2d, dst2d], axis=1)
  src2d1 = jnp.concatenate([src, padi]).reshape(EPAD // K1, K1)
  dst2d1 = jnp.concatenate([dst, padi]).reshape(EPAD // K1, K1)
  eidx1 = jnp.stack([src2d1, dst2d1], axis=1)

  degs = _deg_call(dst2d).reshape(2, NPAD)
  xw = _m0_call(x, W1)
  degT = degs.T
  y1, dis = _m1_call(xw, degT)
  z1 = _a1_call(y1.reshape(2 * NPAD, 128), eidx1)
  y2 = _m2_call(z1, dis, b1.reshape(1, HID), W2)
  zeros = jnp.zeros((NPAD, CLS), jnp.float32)
  parts = _a2_call(y2, eidx, zeros)
  return _m3_call(parts, dis, b2.reshape(1, CLS))
